# Initial kernel scaffold; baseline (speedup 1.0000x reference)
import functools
import math

import jax
import jax.numpy as jnp
from jax import lax
from jax.experimental import pallas as pl
from jax.experimental.pallas import tpu as pltpu
from jax.experimental.pallas import tpu_sc as plsc

F = 32
H = 16
NC = 2
NS = 16
NW = NC * NS
SCALE = 1.0 / math.sqrt(30.0)


def _pick_chunk(cnt):
    for c in range(128, 0, -8):
        if cnt % c == 0:
            return c
    raise ValueError(f"no chunk for {cnt}")


@functools.cache
def _gather_kernel(T, Ep):
    cnt = Ep // NW
    chunk = _pick_chunk(cnt)
    nchunks = cnt // chunk
    mesh = plsc.VectorSubcoreMesh(core_axis_name="c", subcore_axis_name="s")

    def body(table, idx, out, idx_v, rows_v, sem):
        wid = lax.axis_index("s") * NC + lax.axis_index("c")
        base = wid * cnt

        def step(j, carry):
            off = base + j * chunk
            pltpu.sync_copy(idx.at[pl.ds(off, chunk)], idx_v)
            pltpu.async_copy(table.at[idx_v], rows_v, sem).wait()
            pltpu.sync_copy(rows_v, out.at[pl.ds(off, chunk)])
            return carry

        lax.fori_loop(0, nchunks, step, 0)

    return pl.kernel(
        body,
        out_type=jax.ShapeDtypeStruct((Ep, F), jnp.float32),
        mesh=mesh,
        scratch_types=[
            pltpu.VMEM((chunk,), jnp.int32),
            pltpu.VMEM((chunk, F), jnp.float32),
            pltpu.SemaphoreType.DMA,
        ],
    )


def _gather_rows(table, idx):
    return _gather_kernel(table.shape[0], idx.shape[0])(table, idx)


@functools.cache
def _scatter_kernel(Ep, Tr, ranges):
    cnt = Ep // NS
    chunk = _pick_chunk(cnt)
    nchunks = cnt // chunk
    Tacc = Tr + 32
    wr = Tr // NS
    zr = Tacc // NS
    mesh = plsc.VectorSubcoreMesh(core_axis_name="c", subcore_axis_name="s")

    def body(vals, idx, zeros_hbm, out, idx_v, midx_v, vb, acc):
        c = lax.axis_index("c")
        s = lax.axis_index("s")
        base = s * cnt
        for p in range(ranges):
            rbase = p * Tr
            pltpu.sync_copy(zeros_hbm.at[pl.ds(s * zr, zr)],
                            acc.at[pl.ds(s * zr, zr)])
            plsc.subcore_barrier()

            def step(j, carry):
                off = base + j * chunk
                pltpu.sync_copy(idx.at[pl.ds(off, chunk)], idx_v)
                for kk in range(chunk // 16):
                    iv = idx_v[pl.ds(kk * 16, 16)]
                    rel = iv - rbase
                    ok = (rel >= 0) & (rel < Tr)
                    midx_v[pl.ds(kk * 16, 16)] = jnp.where(ok, rel, Tr)
                pltpu.sync_copy(vals.at[pl.ds(off, chunk), pl.ds(c * H, H)], vb)
                pltpu.sync_copy(vb, acc.at[midx_v], add=True)
                return carry

            lax.fori_loop(0, nchunks, step, 0)
            plsc.subcore_barrier()
            pltpu.sync_copy(acc.at[pl.ds(s * wr, wr)],
                            out.at[pl.ds(rbase + s * wr, wr), pl.ds(c * H, H)])
            plsc.subcore_barrier()

    return pl.kernel(
        body,
        out_type=jax.ShapeDtypeStruct((Tr * ranges, F), jnp.float32),
        mesh=mesh,
        scratch_types=[
            pltpu.VMEM((chunk,), jnp.int32),
            pltpu.VMEM((chunk,), jnp.int32),
            pltpu.VMEM((chunk, H), jnp.float32),
            pltpu.VMEM_SHARED((Tacc, H), jnp.float32),
        ],
    )


def _scatter_rows(vals, idx, Tr, ranges):
    zeros_hbm = jnp.zeros((Tr + 32, H), jnp.float32)
    return _scatter_kernel(idx.shape[0], Tr, ranges)(vals, idx, zeros_hbm)


def _init_body(gf_ref, inc_ref, wi_ref, bt_ref, wg_ref, b_ref, o_ref):
    ip = jnp.dot(inc_ref[...], wi_ref[...], preferred_element_type=jnp.float32)
    oh = (bt_ref[...] == lax.broadcasted_iota(jnp.int32, (1, 64), 1)
          ).astype(jnp.float32)
    y = (jnp.dot(gf_ref[...], wg_ref[...], preferred_element_type=jnp.float32)
         + jnp.dot(oh, ip, preferred_element_type=jnp.float32) + b_ref[...])
    o_ref[...] = jnp.maximum(y, 0.0)


@functools.cache
def _tc_init_kernel(T, blk):
    grid = T // blk
    return pl.pallas_call(
        _init_body,
        grid=(grid,),
        in_specs=[
            pl.BlockSpec((blk, 8), lambda i: (i, 0)),
            pl.BlockSpec((64, 8), lambda i: (0, 0)),
            pl.BlockSpec((8, F), lambda i: (0, 0)),
            pl.BlockSpec((blk, 1), lambda i: (i, 0)),
            pl.BlockSpec((8, F), lambda i: (0, 0)),
            pl.BlockSpec((1, F), lambda i: (0, 0)),
        ],
        out_specs=pl.BlockSpec((blk, F), lambda i: (i, 0)),
        out_shape=jax.ShapeDtypeStruct((T, F), jnp.float32),
    )


def _qkvr_body(x_ref, wq, wk, wv, wr, b_ref, q_ref, k_ref, v_ref, r_ref):
    x = x_ref[...]
    b = b_ref[...]
    q_ref[...] = jnp.dot(x, wq[...], preferred_element_type=jnp.float32) + b[:, 0:F]
    k_ref[...] = jnp.dot(x, wk[...], preferred_element_type=jnp.float32) + b[:, F:2 * F]
    v_ref[...] = jnp.dot(x, wv[...], preferred_element_type=jnp.float32) + b[:, 2 * F:3 * F]
    r_ref[...] = jnp.dot(x, wr[...], preferred_element_type=jnp.float32) + b[:, 3 * F:4 * F]


@functools.cache
def _tc_qkvr_kernel(T, Fin, blk):
    grid = T // blk
    o = jax.ShapeDtypeStruct((T, F), jnp.float32)
    return pl.pallas_call(
        _qkvr_body,
        grid=(grid,),
        in_specs=[pl.BlockSpec((blk, Fin), lambda i: (i, 0))]
        + [pl.BlockSpec((Fin, F), lambda i: (0, 0))] * 4
        + [pl.BlockSpec((1, 4 * F), lambda i: (0, 0))],
        out_specs=[pl.BlockSpec((blk, F), lambda i: (i, 0))] * 4,
        out_shape=[o, o, o, o],
    )


def _amax_body(qd_ref, ks_ref, m_ref):
    i = pl.program_id(0)
    a = jnp.sum(qd_ref[...] * ks_ref[...], axis=1) * SCALE
    mx = jnp.max(a)

    @pl.when(i == 0)
    def _():
        m_ref[0, 0] = mx

    @pl.when(i > 0)
    def _():
        m_ref[0, 0] = jnp.maximum(m_ref[0, 0], mx)


@functools.cache
def _tc_amax_kernel(Ep, blk):
    grid = Ep // blk
    return pl.pallas_call(
        _amax_body,
        grid=(grid,),
        in_specs=[pl.BlockSpec((blk, F), lambda i: (i, 0))] * 2,
        out_specs=pl.BlockSpec((1, 1), lambda i: (0, 0)),
        out_shape=jax.ShapeDtypeStruct((1, 1), jnp.float32),
    )


def _exws_body(qd_ref, ks_ref, vs_ref, m_ref, w_ref):
    a = jnp.sum(qd_ref[...] * ks_ref[...], axis=1, keepdims=True) * SCALE
    ex = jnp.exp(a - m_ref[0, 0])
    col = lax.broadcasted_iota(jnp.int32, w_ref.shape, 1)
    w_ref[...] = vs_ref[...] * ex + jnp.where(col == F - 1, ex, 0.0)


@functools.cache
def _tc_exws_kernel(Ep, blk):
    grid = Ep // blk
    return pl.pallas_call(
        _exws_body,
        grid=(grid,),
        in_specs=[pl.BlockSpec((blk, F), lambda i: (i, 0))] * 3
        + [pl.BlockSpec((1, 1), lambda i: (0, 0))],
        out_specs=pl.BlockSpec((blk, F), lambda i: (i, 0)),
        out_shape=jax.ShapeDtypeStruct((Ep, F), jnp.float32),
    )


def _combine_body(us_ref, r_ref, w1, w2, w3, o_ref):
    u = us_ref[...]
    s = u[:, F - 1:F]
    col = lax.broadcasted_iota(jnp.int32, u.shape, 1)
    out = jnp.where(col >= F - 2, 0.0, u / (s + 1e-16))
    r = r_ref[...]
    lg = jnp.sum(out * w1[...] + r * w2[...] + (out - r) * w3[...],
                 axis=1, keepdims=True)
    beta = 1.0 / (1.0 + jnp.exp(-lg))
    o_ref[...] = jnp.maximum(beta * r + (1.0 - beta) * out, 0.0)


def _combine_final_body(us_ref, r_ref, w1, w2, w3, wf, bf, o_ref):
    u = us_ref[...]
    s = u[:, F - 1:F]
    col = lax.broadcasted_iota(jnp.int32, u.shape, 1)
    out = jnp.where(col >= F - 2, 0.0, u / (s + 1e-16))
    r = r_ref[...]
    lg = jnp.sum(out * w1[...] + r * w2[...] + (out - r) * w3[...],
                 axis=1, keepdims=True)
    beta = 1.0 / (1.0 + jnp.exp(-lg))
    x = jnp.maximum(beta * r + (1.0 - beta) * out, 0.0)
    y = jnp.dot(x, wf[...], preferred_element_type=jnp.float32) + bf[...]
    o_ref[...] = jnp.maximum(y, 0.0)


@functools.cache
def _tc_combine_kernel(T, blk, with_final):
    grid = T // blk
    specs = [pl.BlockSpec((blk, F), lambda i: (i, 0))] * 2 \
        + [pl.BlockSpec((1, F), lambda i: (0, 0))] * 3
    body = _combine_body
    if with_final:
        specs += [pl.BlockSpec((F, F), lambda i: (0, 0)),
                  pl.BlockSpec((1, F), lambda i: (0, 0))]
        body = _combine_final_body
    return pl.pallas_call(
        body,
        grid=(grid,),
        in_specs=specs,
        out_specs=pl.BlockSpec((blk, F), lambda i: (i, 0)),
        out_shape=jax.ShapeDtypeStruct((T, F), jnp.float32),
    )


def _scalemul_body(b_ref, v_ref, o_ref):
    o_ref[...] = b_ref[...] * v_ref[...]


@functools.cache
def _tc_scalemul_kernel(T, blk):
    return pl.pallas_call(
        _scalemul_body,
        grid=(T // blk,),
        in_specs=[pl.BlockSpec((blk, F), lambda i: (i, 0)),
                  pl.BlockSpec((blk, 1), lambda i: (i, 0))],
        out_specs=pl.BlockSpec((blk, F), lambda i: (i, 0)),
        out_shape=jax.ShapeDtypeStruct((T, F), jnp.float32),
    )


def _pool_body(x_ref, bt_ref, s_ref, c_ref):
    i = pl.program_id(0)

    @pl.when(i == 0)
    def _():
        s_ref[...] = jnp.zeros_like(s_ref)
        c_ref[...] = jnp.zeros_like(c_ref)

    oh = (bt_ref[...] == lax.broadcasted_iota(jnp.int32, (1, 64), 1)
          ).astype(jnp.float32)
    s_ref[...] += lax.dot_general(oh, x_ref[...], (((0,), (0,)), ((), ())),
                                  preferred_element_type=jnp.float32)
    c_ref[...] += lax.dot_general(oh, jnp.ones_like(bt_ref[...], jnp.float32),
                                  (((0,), (0,)), ((), ())),
                                  preferred_element_type=jnp.float32)


@functools.cache
def _tc_pool_kernel(T, blk):
    return pl.pallas_call(
        _pool_body,
        grid=(T // blk,),
        in_specs=[pl.BlockSpec((blk, F), lambda i: (i, 0)),
                  pl.BlockSpec((blk, 1), lambda i: (i, 0))],
        out_specs=[pl.BlockSpec((64, F), lambda i: (0, 0)),
                   pl.BlockSpec((64, 1), lambda i: (0, 0))],
        out_shape=[jax.ShapeDtypeStruct((64, F), jnp.float32),
                   jax.ShapeDtypeStruct((64, 1), jnp.float32)],
    )


def _head_body(s_ref, c_ref, w_ref, b_ref, o_ref):
    m = s_ref[...] / jnp.maximum(c_ref[...], 1.0)
    v = jnp.sum(m * w_ref[...], axis=1, keepdims=True) + b_ref[0, 0]
    o_ref[...] = jnp.tanh(v)


@functools.cache
def _tc_head_kernel():
    return pl.pallas_call(
        _head_body,
        grid=(1,),
        in_specs=[pl.BlockSpec((64, F), lambda i: (0, 0)),
                  pl.BlockSpec((64, 1), lambda i: (0, 0)),
                  pl.BlockSpec((1, F), lambda i: (0, 0)),
                  pl.BlockSpec((1, 1), lambda i: (0, 0))],
        out_specs=pl.BlockSpec((64, 1), lambda i: (0, 0)),
        out_shape=jax.ShapeDtypeStruct((64, 1), jnp.float32),
    )


def _pad_w(w):
    din, dout = w.shape
    dp = 32 if din <= 32 else 64
    return jnp.pad(w, ((0, dp - din), (0, F - dout)))


def _pad_vec(b):
    return jnp.pad(b, (0, F - b.shape[0])).reshape(1, F)


def _conv(p, x, src, dst, Tr, ranges, eblk, final=None):
    T, Fin = x.shape
    blk = 2000 if T == 100000 else 2048
    wq, wk, wv, wr = (_pad_w(p[n]["w"]) for n in ("q", "k", "v", "skip"))
    bc = jnp.concatenate(
        [_pad_vec(p[n]["b"]) for n in ("q", "k", "v", "skip")], axis=1)
    q, k, v, r = _tc_qkvr_kernel(T, Fin, blk)(x, wq, wk, wv, wr, bc)
    qd = _gather_rows(q, dst)
    ks = _gather_rows(k, src)
    vs = _gather_rows(v, src)
    Ep = src.shape[0]
    m = _tc_amax_kernel(Ep, eblk)(qd, ks)
    ws = _tc_exws_kernel(Ep, eblk)(qd, ks, vs, m)
    us = _scatter_rows(ws, dst, Tr, ranges)[:T]
    bw = p["beta_w"][:, 0]
    w1 = _pad_vec(bw[0:30])
    w2 = _pad_vec(bw[30:60])
    w3 = _pad_vec(bw[60:90])
    if final is None:
        return _tc_combine_kernel(T, blk, False)(us, r, w1, w2, w3)
    wf, bf = final
    return _tc_combine_kernel(T, blk, True)(us, r, w1, w2, w3, wf, bf)


@jax.jit
def _impl(graph_features, income, bonus_values_normed, batch, graph_edges,
          bonus_nodes, bonus_edges, bonus_batch, bonus_mapping, params):
    N = graph_features.shape[0]
    NB = bonus_nodes.shape[0]
    NBON = bonus_values_normed.shape[0]
    M = bonus_mapping.shape[1]
    NBpad = 200704
    p = params

    gf8 = jnp.pad(graph_features, ((0, 0), (0, 3)))
    inc8 = jnp.pad(income, ((0, 0), (0, 6)))
    wg = jnp.pad(p["init"]["w"][:5], ((0, 3), (0, 2)))
    wi = jnp.pad(p["init"]["w"][5:7], ((0, 6), (0, 2)))
    bi = _pad_vec(p["init"]["b"])
    bt2 = batch.astype(jnp.int32).reshape(N, 1)
    x = _tc_init_kernel(N, 2000)(gf8, inc8, wi, bt2, wg, bi)

    src = graph_edges[0].astype(jnp.int32)
    dst = graph_edges[1].astype(jnp.int32)
    x = _conv(p["g1"], x, src, dst, N, 1, 6400)
    x = _conv(p["g2"], x, src, dst, N, 1, 6400)

    bn = jnp.pad(bonus_nodes.astype(jnp.int32), (0, NBpad - NB))
    xb = _gather_rows(x, bn)
    bsrc = bonus_edges[0].astype(jnp.int32)
    bdst = bonus_edges[1].astype(jnp.int32)
    xb = _conv(p["b1"], xb, bsrc, bdst, NBpad // 2, 2, 6400)
    bb = jnp.pad(bonus_batch.astype(jnp.int32), (0, NBpad - NB),
                 constant_values=NBON)
    pooled = _scatter_rows(xb, bb, NBON, 1)
    c2 = _tc_scalemul_kernel(NBON, 2000)(
        pooled, bonus_values_normed.reshape(NBON, 1))
    cols = jnp.pad(bonus_mapping[1].astype(jnp.int32), (0, NBpad - M))
    rows = jnp.pad(bonus_mapping[0].astype(jnp.int32), (0, NBpad - M),
                   constant_values=NBON)
    gs = _gather_rows(c2, cols)
    bn20 = _scatter_rows(gs, rows, NBON, 1)
    bnode = jnp.concatenate(
        [bn20, jnp.zeros((N - NBON, F), jnp.float32)], axis=0)

    x3 = jnp.concatenate([x, bnode], axis=1)
    wf1 = _pad_w(p["final1"]["w"])
    bf1 = _pad_vec(p["final1"]["b"])
    x4 = _conv(p["g3"], x3, src, dst, N, 1, 6400, final=(wf1, bf1))

    sums, counts = _tc_pool_kernel(N, 2000)(x4, bt2)
    w2v = _pad_vec(p["final2"]["w"][:, 0])
    b2s = p["final2"]["b"].reshape(1, 1)
    out = _tc_head_kernel()(sums, counts, w2v, b2s)
    return out.reshape(-1)


def kernel(graph_features, income, bonus_values_normed, batch, graph_edges,
           bonus_nodes, bonus_edges, bonus_batch, bonus_mapping, params):
    return _impl(graph_features, income, bonus_values_normed, batch,
                 graph_edges, bonus_nodes, bonus_edges, bonus_batch,
                 bonus_mapping, params)


# speedup vs baseline: 5.7552x; 5.7552x over previous
import functools
import math

import jax
import jax.numpy as jnp
from jax import lax
from jax.experimental import pallas as pl
from jax.experimental.pallas import tpu as pltpu
from jax.experimental.pallas import tpu_sc as plsc

F = 32
H = 16
NC = 2
NS = 16
NW = NC * NS
SCALE = 1.0 / math.sqrt(30.0)


def _pick_chunk(cnt):
    for c in range(128, 0, -8):
        if cnt % c == 0:
            return c
    raise ValueError(f"no chunk for {cnt}")


@functools.cache
def _gather_kernel(T, Ep):
    cnt = Ep // NW
    chunk = _pick_chunk(cnt)
    nchunks = cnt // chunk
    mesh = plsc.VectorSubcoreMesh(core_axis_name="c", subcore_axis_name="s")

    def body(table, idx, out, idx_v, rows_v, sem):
        wid = lax.axis_index("s") * NC + lax.axis_index("c")
        base = wid * cnt

        def step(j, carry):
            off = base + j * chunk
            pltpu.sync_copy(idx.at[pl.ds(off, chunk)], idx_v)
            pltpu.async_copy(table.at[idx_v], rows_v, sem).wait()
            pltpu.sync_copy(rows_v, out.at[pl.ds(off, chunk)])
            return carry

        lax.fori_loop(0, nchunks, step, 0)

    return pl.kernel(
        body,
        out_type=jax.ShapeDtypeStruct((Ep, F), jnp.float32),
        mesh=mesh,
        compiler_params=pltpu.CompilerParams(use_tc_tiling_on_sc=False),
        scratch_types=[
            pltpu.VMEM((chunk,), jnp.int32),
            pltpu.VMEM((chunk, F), jnp.float32),
            pltpu.SemaphoreType.DMA,
        ],
    )


def _gather_rows(table, idx):
    return _gather_kernel(table.shape[0], idx.shape[0])(table, idx)


@functools.cache
def _scatter_kernel(Ep, Tr, ranges):
    cnt = Ep // NS
    chunk = _pick_chunk(cnt)
    nchunks = cnt // chunk
    Tacc = Tr + 32
    wr = Tr // NS
    zr = Tacc // NS
    mesh = plsc.VectorSubcoreMesh(core_axis_name="c", subcore_axis_name="s")

    def body(vals, idx, zeros_hbm, out, idx_v, midx_v, vb, acc):
        c = lax.axis_index("c")
        s = lax.axis_index("s")
        base = s * cnt
        for p in range(ranges):
            rbase = p * Tr
            pltpu.sync_copy(zeros_hbm.at[pl.ds(s * zr, zr)],
                            acc.at[pl.ds(s * zr, zr)])
            plsc.subcore_barrier()

            def step(j, carry):
                off = base + j * chunk
                pltpu.sync_copy(idx.at[pl.ds(off, chunk)], idx_v)
                for kk in range(chunk // 16):
                    iv = idx_v[pl.ds(kk * 16, 16)]
                    rel = iv - rbase
                    ok = (rel >= 0) & (rel < Tr)
                    midx_v[pl.ds(kk * 16, 16)] = jnp.where(ok, rel, Tr)
                pltpu.sync_copy(vals.at[pl.ds(off, chunk), pl.ds(c * H, H)], vb)
                pltpu.sync_copy(vb, acc.at[midx_v], add=True)
                return carry

            lax.fori_loop(0, nchunks, step, 0)
            plsc.subcore_barrier()
            pltpu.sync_copy(acc.at[pl.ds(s * wr, wr)],
                            out.at[pl.ds(rbase + s * wr, wr), pl.ds(c * H, H)])
            plsc.subcore_barrier()

    return pl.kernel(
        body,
        out_type=jax.ShapeDtypeStruct((Tr * ranges, F), jnp.float32),
        mesh=mesh,
        compiler_params=pltpu.CompilerParams(use_tc_tiling_on_sc=False),
        scratch_types=[
            pltpu.VMEM((chunk,), jnp.int32),
            pltpu.VMEM((chunk,), jnp.int32),
            pltpu.VMEM((chunk, H), jnp.float32),
            pltpu.VMEM_SHARED((Tacc, H), jnp.float32),
        ],
    )


def _scatter_rows(vals, idx, Tr, ranges):
    zeros_hbm = jnp.zeros((Tr + 32, H), jnp.float32)
    return _scatter_kernel(idx.shape[0], Tr, ranges)(vals, idx, zeros_hbm)


def _init_body(gf_ref, inc_ref, wi_ref, bt_ref, wg_ref, b_ref, o_ref):
    ip = jnp.dot(inc_ref[...], wi_ref[...], preferred_element_type=jnp.float32)
    oh = (bt_ref[...] == lax.broadcasted_iota(jnp.int32, (1, 64), 1)
          ).astype(jnp.float32)
    y = (jnp.dot(gf_ref[...], wg_ref[...], preferred_element_type=jnp.float32)
         + jnp.dot(oh, ip, preferred_element_type=jnp.float32) + b_ref[...])
    o_ref[...] = jnp.maximum(y, 0.0)


@functools.cache
def _tc_init_kernel(T, blk):
    grid = T // blk
    return pl.pallas_call(
        _init_body,
        grid=(grid,),
        in_specs=[
            pl.BlockSpec((blk, 8), lambda i: (i, 0)),
            pl.BlockSpec((64, 8), lambda i: (0, 0)),
            pl.BlockSpec((8, F), lambda i: (0, 0)),
            pl.BlockSpec((blk, 1), lambda i: (i, 0)),
            pl.BlockSpec((8, F), lambda i: (0, 0)),
            pl.BlockSpec((1, F), lambda i: (0, 0)),
        ],
        out_specs=pl.BlockSpec((blk, F), lambda i: (i, 0)),
        out_shape=jax.ShapeDtypeStruct((T, F), jnp.float32),
    )


def _qkvr_body(x_ref, wq, wk, wv, wr, b_ref, q_ref, k_ref, v_ref, r_ref):
    x = x_ref[...]
    b = b_ref[...]
    q_ref[...] = jnp.dot(x, wq[...], preferred_element_type=jnp.float32) + b[:, 0:F]
    k_ref[...] = jnp.dot(x, wk[...], preferred_element_type=jnp.float32) + b[:, F:2 * F]
    v_ref[...] = jnp.dot(x, wv[...], preferred_element_type=jnp.float32) + b[:, 2 * F:3 * F]
    r_ref[...] = jnp.dot(x, wr[...], preferred_element_type=jnp.float32) + b[:, 3 * F:4 * F]


@functools.cache
def _tc_qkvr_kernel(T, Fin, blk):
    grid = T // blk
    o = jax.ShapeDtypeStruct((T, F), jnp.float32)
    return pl.pallas_call(
        _qkvr_body,
        grid=(grid,),
        in_specs=[pl.BlockSpec((blk, Fin), lambda i: (i, 0))]
        + [pl.BlockSpec((Fin, F), lambda i: (0, 0))] * 4
        + [pl.BlockSpec((1, 4 * F), lambda i: (0, 0))],
        out_specs=[pl.BlockSpec((blk, F), lambda i: (i, 0))] * 4,
        out_shape=[o, o, o, o],
    )


def _amax_body(qd_ref, ks_ref, m_ref):
    i = pl.program_id(0)
    a = jnp.sum(qd_ref[...] * ks_ref[...], axis=1) * SCALE
    mx = jnp.max(a)

    @pl.when(i == 0)
    def _():
        m_ref[0, 0] = mx

    @pl.when(i > 0)
    def _():
        m_ref[0, 0] = jnp.maximum(m_ref[0, 0], mx)


@functools.cache
def _tc_amax_kernel(Ep, blk):
    grid = Ep // blk
    return pl.pallas_call(
        _amax_body,
        grid=(grid,),
        in_specs=[pl.BlockSpec((blk, F), lambda i: (i, 0))] * 2,
        out_specs=pl.BlockSpec(memory_space=pltpu.SMEM),
        out_shape=jax.ShapeDtypeStruct((1, 1), jnp.float32),
    )


def _exws_body(qd_ref, ks_ref, vs_ref, m_ref, w_ref):
    a = jnp.sum(qd_ref[...] * ks_ref[...], axis=1, keepdims=True) * SCALE
    ex = jnp.exp(a - m_ref[0, 0])
    col = lax.broadcasted_iota(jnp.int32, w_ref.shape, 1)
    w_ref[...] = vs_ref[...] * ex + jnp.where(col == F - 1, ex, 0.0)


@functools.cache
def _tc_exws_kernel(Ep, blk):
    grid = Ep // blk
    return pl.pallas_call(
        _exws_body,
        grid=(grid,),
        in_specs=[pl.BlockSpec((blk, F), lambda i: (i, 0))] * 3
        + [pl.BlockSpec(memory_space=pltpu.SMEM)],
        out_specs=pl.BlockSpec((blk, F), lambda i: (i, 0)),
        out_shape=jax.ShapeDtypeStruct((Ep, F), jnp.float32),
    )


def _combine_body(us_ref, r_ref, w1, w2, w3, o_ref):
    u = us_ref[...]
    s = u[:, F - 1:F]
    col = lax.broadcasted_iota(jnp.int32, u.shape, 1)
    out = jnp.where(col >= F - 2, 0.0, u / (s + 1e-16))
    r = r_ref[...]
    lg = jnp.sum(out * w1[...] + r * w2[...] + (out - r) * w3[...],
                 axis=1, keepdims=True)
    beta = 1.0 / (1.0 + jnp.exp(-lg))
    o_ref[...] = jnp.maximum(beta * r + (1.0 - beta) * out, 0.0)


def _combine_final_body(us_ref, r_ref, w1, w2, w3, wf, bf, o_ref):
    u = us_ref[...]
    s = u[:, F - 1:F]
    col = lax.broadcasted_iota(jnp.int32, u.shape, 1)
    out = jnp.where(col >= F - 2, 0.0, u / (s + 1e-16))
    r = r_ref[...]
    lg = jnp.sum(out * w1[...] + r * w2[...] + (out - r) * w3[...],
                 axis=1, keepdims=True)
    beta = 1.0 / (1.0 + jnp.exp(-lg))
    x = jnp.maximum(beta * r + (1.0 - beta) * out, 0.0)
    y = jnp.dot(x, wf[...], preferred_element_type=jnp.float32) + bf[...]
    o_ref[...] = jnp.maximum(y, 0.0)


@functools.cache
def _tc_combine_kernel(T, blk, with_final):
    grid = T // blk
    specs = [pl.BlockSpec((blk, F), lambda i: (i, 0))] * 2 \
        + [pl.BlockSpec((1, F), lambda i: (0, 0))] * 3
    body = _combine_body
    if with_final:
        specs += [pl.BlockSpec((F, F), lambda i: (0, 0)),
                  pl.BlockSpec((1, F), lambda i: (0, 0))]
        body = _combine_final_body
    return pl.pallas_call(
        body,
        grid=(grid,),
        in_specs=specs,
        out_specs=pl.BlockSpec((blk, F), lambda i: (i, 0)),
        out_shape=jax.ShapeDtypeStruct((T, F), jnp.float32),
    )


def _scalemul_body(b_ref, v_ref, o_ref):
    o_ref[...] = b_ref[...] * v_ref[...]


@functools.cache
def _tc_scalemul_kernel(T, blk):
    return pl.pallas_call(
        _scalemul_body,
        grid=(T // blk,),
        in_specs=[pl.BlockSpec((blk, F), lambda i: (i, 0)),
                  pl.BlockSpec((blk, 1), lambda i: (i, 0))],
        out_specs=pl.BlockSpec((blk, F), lambda i: (i, 0)),
        out_shape=jax.ShapeDtypeStruct((T, F), jnp.float32),
    )


def _pool_body(x_ref, bt_ref, s_ref, c_ref):
    i = pl.program_id(0)

    @pl.when(i == 0)
    def _():
        s_ref[...] = jnp.zeros_like(s_ref)
        c_ref[...] = jnp.zeros_like(c_ref)

    oh = (bt_ref[...] == lax.broadcasted_iota(jnp.int32, (1, 64), 1)
          ).astype(jnp.float32)
    s_ref[...] += lax.dot_general(oh, x_ref[...], (((0,), (0,)), ((), ())),
                                  preferred_element_type=jnp.float32)
    c_ref[...] += lax.dot_general(oh, jnp.ones_like(bt_ref[...], jnp.float32),
                                  (((0,), (0,)), ((), ())),
                                  preferred_element_type=jnp.float32)


@functools.cache
def _tc_pool_kernel(T, blk):
    return pl.pallas_call(
        _pool_body,
        grid=(T // blk,),
        in_specs=[pl.BlockSpec((blk, F), lambda i: (i, 0)),
                  pl.BlockSpec((blk, 1), lambda i: (i, 0))],
        out_specs=[pl.BlockSpec((64, F), lambda i: (0, 0)),
                   pl.BlockSpec((64, 1), lambda i: (0, 0))],
        out_shape=[jax.ShapeDtypeStruct((64, F), jnp.float32),
                   jax.ShapeDtypeStruct((64, 1), jnp.float32)],
    )


def _head_body(s_ref, c_ref, w_ref, b_ref, o_ref):
    m = s_ref[...] / jnp.maximum(c_ref[...], 1.0)
    v = jnp.sum(m * w_ref[...], axis=1, keepdims=True) + b_ref[0, 0]
    o_ref[...] = jnp.tanh(v)


@functools.cache
def _tc_head_kernel():
    return pl.pallas_call(
        _head_body,
        grid=(1,),
        in_specs=[pl.BlockSpec((64, F), lambda i: (0, 0)),
                  pl.BlockSpec((64, 1), lambda i: (0, 0)),
                  pl.BlockSpec((1, F), lambda i: (0, 0)),
                  pl.BlockSpec((1, 1), lambda i: (0, 0))],
        out_specs=pl.BlockSpec((64, 1), lambda i: (0, 0)),
        out_shape=jax.ShapeDtypeStruct((64, 1), jnp.float32),
    )


def _pad_w(w):
    din, dout = w.shape
    if din > 32:
        h = din // 2
        top = jnp.pad(w[:h], ((0, 32 - h), (0, F - dout)))
        bot = jnp.pad(w[h:], ((0, 32 - (din - h)), (0, F - dout)))
        return jnp.concatenate([top, bot], axis=0)
    return jnp.pad(w, ((0, 32 - din), (0, F - dout)))


def _pad_vec(b):
    return jnp.pad(b, (0, F - b.shape[0])).reshape(1, F)


def _conv(p, x, src, dst, Tr, ranges, eblk, final=None):
    T, Fin = x.shape
    blk = 2000 if T == 100000 else 2048
    wq, wk, wv, wr = (_pad_w(p[n]["w"]) for n in ("q", "k", "v", "skip"))
    bc = jnp.concatenate(
        [_pad_vec(p[n]["b"]) for n in ("q", "k", "v", "skip")], axis=1)
    q, k, v, r = _tc_qkvr_kernel(T, Fin, blk)(x, wq, wk, wv, wr, bc)
    qd = _gather_rows(q, dst)
    ks = _gather_rows(k, src)
    vs = _gather_rows(v, src)
    Ep = src.shape[0]
    m = _tc_amax_kernel(Ep, eblk)(qd, ks)
    ws = _tc_exws_kernel(Ep, eblk)(qd, ks, vs, m)
    us = _scatter_rows(ws, dst, Tr, ranges)[:T]
    bw = p["beta_w"][:, 0]
    w1 = _pad_vec(bw[0:30])
    w2 = _pad_vec(bw[30:60])
    w3 = _pad_vec(bw[60:90])
    if final is None:
        return _tc_combine_kernel(T, blk, False)(us, r, w1, w2, w3)
    wf, bf = final
    return _tc_combine_kernel(T, blk, True)(us, r, w1, w2, w3, wf, bf)


@jax.jit
def _impl(graph_features, income, bonus_values_normed, batch, graph_edges,
          bonus_nodes, bonus_edges, bonus_batch, bonus_mapping, params):
    N = graph_features.shape[0]
    NB = bonus_nodes.shape[0]
    NBON = bonus_values_normed.shape[0]
    M = bonus_mapping.shape[1]
    NBpad = 200704
    p = params

    gf8 = jnp.pad(graph_features, ((0, 0), (0, 3)))
    inc8 = jnp.pad(income, ((0, 0), (0, 6)))
    wg = jnp.pad(p["init"]["w"][:5], ((0, 3), (0, 2)))
    wi = jnp.pad(p["init"]["w"][5:7], ((0, 6), (0, 2)))
    bi = _pad_vec(p["init"]["b"])
    bt2 = batch.astype(jnp.int32).reshape(N, 1)
    x = _tc_init_kernel(N, 2000)(gf8, inc8, wi, bt2, wg, bi)

    src = graph_edges[0].astype(jnp.int32)
    dst = graph_edges[1].astype(jnp.int32)
    x = _conv(p["g1"], x, src, dst, N, 1, 6400)
    x = _conv(p["g2"], x, src, dst, N, 1, 6400)

    bn = jnp.pad(bonus_nodes.astype(jnp.int32), (0, NBpad - NB))
    xb = _gather_rows(x, bn)
    bsrc = bonus_edges[0].astype(jnp.int32)
    bdst = bonus_edges[1].astype(jnp.int32)
    xb = _conv(p["b1"], xb, bsrc, bdst, NBpad // 2, 2, 6400)
    bb = jnp.pad(bonus_batch.astype(jnp.int32), (0, NBpad - NB),
                 constant_values=NBON)
    pooled = _scatter_rows(xb, bb, NBON, 1)
    c2 = _tc_scalemul_kernel(NBON, 2000)(
        pooled, bonus_values_normed.reshape(NBON, 1))
    cols = jnp.pad(bonus_mapping[1].astype(jnp.int32), (0, NBpad - M))
    rows = jnp.pad(bonus_mapping[0].astype(jnp.int32), (0, NBpad - M),
                   constant_values=NBON)
    gs = _gather_rows(c2, cols)
    bn20 = _scatter_rows(gs, rows, NBON, 1)
    bnode = jnp.concatenate(
        [bn20, jnp.zeros((N - NBON, F), jnp.float32)], axis=0)

    x3 = jnp.concatenate([x, bnode], axis=1)
    wf1 = _pad_w(p["final1"]["w"])
    bf1 = _pad_vec(p["final1"]["b"])
    x4 = _conv(p["g3"], x3, src, dst, N, 1, 6400, final=(wf1, bf1))

    sums, counts = _tc_pool_kernel(N, 2000)(x4, bt2)
    w2v = _pad_vec(p["final2"]["w"][:, 0])
    b2s = p["final2"]["b"].reshape(1, 1)
    out = _tc_head_kernel()(sums, counts, w2v, b2s)
    return out.reshape(-1)


def kernel(graph_features, income, bonus_values_normed, batch, graph_edges,
           bonus_nodes, bonus_edges, bonus_batch, bonus_mapping, params):
    return _impl(graph_features, income, bonus_values_normed, batch,
                 graph_edges, bonus_nodes, bonus_edges, bonus_batch,
                 bonus_mapping, params)


# speedup vs baseline: 6.8976x; 1.1985x over previous
import functools
import math

import jax
import jax.numpy as jnp
from jax import lax
from jax.experimental import pallas as pl
from jax.experimental.pallas import tpu as pltpu
from jax.experimental.pallas import tpu_sc as plsc

F = 32
H = 16
NC = 2
NS = 16
NW = NC * NS
SCALE = 1.0 / math.sqrt(30.0)


def _pick_chunk(cnt):
    for c in range(128, 0, -8):
        if cnt % c == 0:
            return c
    raise ValueError(f"no chunk for {cnt}")


def _pick_block(cnt, maxbb=2048):
    for c in (2048, 1024, 512, 256, 128):
        if c <= maxbb and cnt % c == 0:
            return c
    raise ValueError(f"no block for {cnt}")


@functools.cache
def _gather_kernel(T, Ep):
    cnt = Ep // NW
    bb = _pick_block(cnt)
    nsub = bb // 128
    nb = cnt // bb
    mesh = plsc.VectorSubcoreMesh(core_axis_name="c", subcore_axis_name="s")

    def body(table, idx, out, idx_v, rows_v, sem_i, sem_g, sem_w):
        wid = lax.axis_index("s") * NC + lax.axis_index("c")
        base = wid * cnt

        def step(j, carry):
            off = base + j * bb
            pltpu.async_copy(idx.at[pl.ds(off, bb)], idx_v, sem_i).wait()
            descs = [
                pltpu.async_copy(table.at[idx_v.at[pl.ds(k * 128, 128)]],
                                 rows_v.at[pl.ds(k * 128, 128)], sem_g)
                for k in range(nsub)
            ]
            for d in descs:
                d.wait()
            pltpu.async_copy(rows_v, out.at[pl.ds(off, bb)], sem_w).wait()
            return carry

        lax.fori_loop(0, nb, step, 0)

    return pl.kernel(
        body,
        out_type=jax.ShapeDtypeStruct((Ep, F), jnp.float32),
        mesh=mesh,
        compiler_params=pltpu.CompilerParams(use_tc_tiling_on_sc=False),
        scratch_types=[
            pltpu.VMEM((bb,), jnp.int32),
            pltpu.VMEM((bb, F), jnp.float32),
            pltpu.SemaphoreType.DMA,
            pltpu.SemaphoreType.DMA,
            pltpu.SemaphoreType.DMA,
        ],
    )


def _gather_rows(table, idx):
    return _gather_kernel(table.shape[0], idx.shape[0])(table, idx)


@functools.cache
def _scatter_kernel(Ep, Tr, ranges):
    cnt = Ep // NS
    bb = _pick_block(cnt, maxbb=512)
    nsub = bb // 128
    nb = cnt // bb
    Tacc = Tr + 32
    wr = Tr // NS
    zr = Tacc // NS
    mesh = plsc.VectorSubcoreMesh(core_axis_name="c", subcore_axis_name="s")

    def body(vals, idx, zeros_hbm, out, idx_v, midx_v, vb, acc,
             sem_i, sem_v, sem_s):
        c = lax.axis_index("c")
        s = lax.axis_index("s")
        base = s * cnt
        for p in range(ranges):
            rbase = p * Tr
            pltpu.sync_copy(zeros_hbm.at[pl.ds(s * zr, zr)],
                            acc.at[pl.ds(s * zr, zr)])
            plsc.subcore_barrier()

            def step(j, carry):
                off = base + j * bb
                di = pltpu.async_copy(idx.at[pl.ds(off, bb)], idx_v, sem_i)
                dv = pltpu.async_copy(
                    vals.at[pl.ds(off, bb), pl.ds(c * H, H)], vb, sem_v)
                di.wait()
                for kk in range(bb // 16):
                    iv = idx_v[pl.ds(kk * 16, 16)]
                    rel = iv - rbase
                    ok = (rel >= 0) & (rel < Tr)
                    midx_v[kk // 8, pl.ds((kk % 8) * 16, 16)] = \
                        jnp.where(ok, rel, Tr)
                dv.wait()
                for k in range(nsub):
                    pltpu.sync_copy(vb.at[pl.ds(k * 128, 128)],
                                    acc.at[midx_v.at[k]], add=True)
                return carry

            lax.fori_loop(0, nb, step, 0)
            plsc.subcore_barrier()
            pltpu.sync_copy(acc.at[pl.ds(s * wr, wr)],
                            out.at[pl.ds(rbase + s * wr, wr), pl.ds(c * H, H)])
            plsc.subcore_barrier()

    return pl.kernel(
        body,
        out_type=jax.ShapeDtypeStruct((Tr * ranges, F), jnp.float32),
        mesh=mesh,
        compiler_params=pltpu.CompilerParams(use_tc_tiling_on_sc=False),
        scratch_types=[
            pltpu.VMEM((bb,), jnp.int32),
            pltpu.VMEM((nsub, 128), jnp.int32),
            pltpu.VMEM((bb, H), jnp.float32),
            pltpu.VMEM_SHARED((Tacc, H), jnp.float32),
            pltpu.SemaphoreType.DMA,
            pltpu.SemaphoreType.DMA,
            pltpu.SemaphoreType.DMA,
        ],
    )


def _scatter_rows(vals, idx, Tr, ranges):
    zeros_hbm = jnp.zeros((Tr + 32, H), jnp.float32)
    return _scatter_kernel(idx.shape[0], Tr, ranges)(vals, idx, zeros_hbm)


def _init_body(gf_ref, inc_ref, wi_ref, bt_ref, wg_ref, b_ref, o_ref):
    ip = jnp.dot(inc_ref[...], wi_ref[...], preferred_element_type=jnp.float32)
    oh = (bt_ref[...] == lax.broadcasted_iota(jnp.int32, (1, 64), 1)
          ).astype(jnp.float32)
    y = (jnp.dot(gf_ref[...], wg_ref[...], preferred_element_type=jnp.float32)
         + jnp.dot(oh, ip, preferred_element_type=jnp.float32) + b_ref[...])
    o_ref[...] = jnp.maximum(y, 0.0)


@functools.cache
def _tc_init_kernel(T, blk):
    grid = T // blk
    return pl.pallas_call(
        _init_body,
        grid=(grid,),
        in_specs=[
            pl.BlockSpec((blk, 8), lambda i: (i, 0)),
            pl.BlockSpec((64, 8), lambda i: (0, 0)),
            pl.BlockSpec((8, F), lambda i: (0, 0)),
            pl.BlockSpec((blk, 1), lambda i: (i, 0)),
            pl.BlockSpec((8, F), lambda i: (0, 0)),
            pl.BlockSpec((1, F), lambda i: (0, 0)),
        ],
        out_specs=pl.BlockSpec((blk, F), lambda i: (i, 0)),
        out_shape=jax.ShapeDtypeStruct((T, F), jnp.float32),
    )


def _qkvr_body(x_ref, wq, wk, wv, wr, b_ref, q_ref, k_ref, v_ref, r_ref):
    x = x_ref[...]
    b = b_ref[...]
    q_ref[...] = jnp.dot(x, wq[...], preferred_element_type=jnp.float32) + b[:, 0:F]
    k_ref[...] = jnp.dot(x, wk[...], preferred_element_type=jnp.float32) + b[:, F:2 * F]
    v_ref[...] = jnp.dot(x, wv[...], preferred_element_type=jnp.float32) + b[:, 2 * F:3 * F]
    r_ref[...] = jnp.dot(x, wr[...], preferred_element_type=jnp.float32) + b[:, 3 * F:4 * F]


@functools.cache
def _tc_qkvr_kernel(T, Fin, blk):
    grid = T // blk
    o = jax.ShapeDtypeStruct((T, F), jnp.float32)
    return pl.pallas_call(
        _qkvr_body,
        grid=(grid,),
        in_specs=[pl.BlockSpec((blk, Fin), lambda i: (i, 0))]
        + [pl.BlockSpec((Fin, F), lambda i: (0, 0))] * 4
        + [pl.BlockSpec((1, 4 * F), lambda i: (0, 0))],
        out_specs=[pl.BlockSpec((blk, F), lambda i: (i, 0))] * 4,
        out_shape=[o, o, o, o],
    )


def _amax_body(qd_ref, ks_ref, m_ref):
    i = pl.program_id(0)
    a = jnp.sum(qd_ref[...] * ks_ref[...], axis=1) * SCALE
    mx = jnp.max(a)

    @pl.when(i == 0)
    def _():
        m_ref[0, 0] = mx

    @pl.when(i > 0)
    def _():
        m_ref[0, 0] = jnp.maximum(m_ref[0, 0], mx)


@functools.cache
def _tc_amax_kernel(Ep, blk):
    grid = Ep // blk
    return pl.pallas_call(
        _amax_body,
        grid=(grid,),
        in_specs=[pl.BlockSpec((blk, F), lambda i: (i, 0))] * 2,
        out_specs=pl.BlockSpec(memory_space=pltpu.SMEM),
        out_shape=jax.ShapeDtypeStruct((1, 1), jnp.float32),
    )


def _exws_body(qd_ref, ks_ref, vs_ref, m_ref, w_ref):
    a = jnp.sum(qd_ref[...] * ks_ref[...], axis=1, keepdims=True) * SCALE
    ex = jnp.exp(a - m_ref[0, 0])
    col = lax.broadcasted_iota(jnp.int32, w_ref.shape, 1)
    w_ref[...] = vs_ref[...] * ex + jnp.where(col == F - 1, ex, 0.0)


@functools.cache
def _tc_exws_kernel(Ep, blk):
    grid = Ep // blk
    return pl.pallas_call(
        _exws_body,
        grid=(grid,),
        in_specs=[pl.BlockSpec((blk, F), lambda i: (i, 0))] * 3
        + [pl.BlockSpec(memory_space=pltpu.SMEM)],
        out_specs=pl.BlockSpec((blk, F), lambda i: (i, 0)),
        out_shape=jax.ShapeDtypeStruct((Ep, F), jnp.float32),
    )


def _combine_body(us_ref, r_ref, w1, w2, w3, o_ref):
    u = us_ref[...]
    s = u[:, F - 1:F]
    col = lax.broadcasted_iota(jnp.int32, u.shape, 1)
    out = jnp.where(col >= F - 2, 0.0, u / (s + 1e-16))
    r = r_ref[...]
    lg = jnp.sum(out * w1[...] + r * w2[...] + (out - r) * w3[...],
                 axis=1, keepdims=True)
    beta = 1.0 / (1.0 + jnp.exp(-lg))
    o_ref[...] = jnp.maximum(beta * r + (1.0 - beta) * out, 0.0)


def _combine_final_body(us_ref, r_ref, w1, w2, w3, wf, bf, o_ref):
    u = us_ref[...]
    s = u[:, F - 1:F]
    col = lax.broadcasted_iota(jnp.int32, u.shape, 1)
    out = jnp.where(col >= F - 2, 0.0, u / (s + 1e-16))
    r = r_ref[...]
    lg = jnp.sum(out * w1[...] + r * w2[...] + (out - r) * w3[...],
                 axis=1, keepdims=True)
    beta = 1.0 / (1.0 + jnp.exp(-lg))
    x = jnp.maximum(beta * r + (1.0 - beta) * out, 0.0)
    y = jnp.dot(x, wf[...], preferred_element_type=jnp.float32) + bf[...]
    o_ref[...] = jnp.maximum(y, 0.0)


@functools.cache
def _tc_combine_kernel(T, blk, with_final):
    grid = T // blk
    specs = [pl.BlockSpec((blk, F), lambda i: (i, 0))] * 2 \
        + [pl.BlockSpec((1, F), lambda i: (0, 0))] * 3
    body = _combine_body
    if with_final:
        specs += [pl.BlockSpec((F, F), lambda i: (0, 0)),
                  pl.BlockSpec((1, F), lambda i: (0, 0))]
        body = _combine_final_body
    return pl.pallas_call(
        body,
        grid=(grid,),
        in_specs=specs,
        out_specs=pl.BlockSpec((blk, F), lambda i: (i, 0)),
        out_shape=jax.ShapeDtypeStruct((T, F), jnp.float32),
    )


def _scalemul_body(b_ref, v_ref, o_ref):
    o_ref[...] = b_ref[...] * v_ref[...]


@functools.cache
def _tc_scalemul_kernel(T, blk):
    return pl.pallas_call(
        _scalemul_body,
        grid=(T // blk,),
        in_specs=[pl.BlockSpec((blk, F), lambda i: (i, 0)),
                  pl.BlockSpec((blk, 1), lambda i: (i, 0))],
        out_specs=pl.BlockSpec((blk, F), lambda i: (i, 0)),
        out_shape=jax.ShapeDtypeStruct((T, F), jnp.float32),
    )


def _pool_body(x_ref, bt_ref, s_ref, c_ref):
    i = pl.program_id(0)

    @pl.when(i == 0)
    def _():
        s_ref[...] = jnp.zeros_like(s_ref)
        c_ref[...] = jnp.zeros_like(c_ref)

    oh = (bt_ref[...] == lax.broadcasted_iota(jnp.int32, (1, 64), 1)
          ).astype(jnp.float32)
    s_ref[...] += lax.dot_general(oh, x_ref[...], (((0,), (0,)), ((), ())),
                                  preferred_element_type=jnp.float32)
    c_ref[...] += lax.dot_general(oh, jnp.ones_like(bt_ref[...], jnp.float32),
                                  (((0,), (0,)), ((), ())),
                                  preferred_element_type=jnp.float32)


@functools.cache
def _tc_pool_kernel(T, blk):
    return pl.pallas_call(
        _pool_body,
        grid=(T // blk,),
        in_specs=[pl.BlockSpec((blk, F), lambda i: (i, 0)),
                  pl.BlockSpec((blk, 1), lambda i: (i, 0))],
        out_specs=[pl.BlockSpec((64, F), lambda i: (0, 0)),
                   pl.BlockSpec((64, 1), lambda i: (0, 0))],
        out_shape=[jax.ShapeDtypeStruct((64, F), jnp.float32),
                   jax.ShapeDtypeStruct((64, 1), jnp.float32)],
    )


def _head_body(s_ref, c_ref, w_ref, b_ref, o_ref):
    m = s_ref[...] / jnp.maximum(c_ref[...], 1.0)
    v = jnp.sum(m * w_ref[...], axis=1, keepdims=True) + b_ref[0, 0]
    o_ref[...] = jnp.tanh(v)


@functools.cache
def _tc_head_kernel():
    return pl.pallas_call(
        _head_body,
        grid=(1,),
        in_specs=[pl.BlockSpec((64, F), lambda i: (0, 0)),
                  pl.BlockSpec((64, 1), lambda i: (0, 0)),
                  pl.BlockSpec((1, F), lambda i: (0, 0)),
                  pl.BlockSpec((1, 1), lambda i: (0, 0))],
        out_specs=pl.BlockSpec((64, 1), lambda i: (0, 0)),
        out_shape=jax.ShapeDtypeStruct((64, 1), jnp.float32),
    )


def _pad_w(w):
    din, dout = w.shape
    if din > 32:
        h = din // 2
        top = jnp.pad(w[:h], ((0, 32 - h), (0, F - dout)))
        bot = jnp.pad(w[h:], ((0, 32 - (din - h)), (0, F - dout)))
        return jnp.concatenate([top, bot], axis=0)
    return jnp.pad(w, ((0, 32 - din), (0, F - dout)))


def _pad_vec(b):
    return jnp.pad(b, (0, F - b.shape[0])).reshape(1, F)


def _conv(p, x, src, dst_g, dst_s, Tr, ranges, eblk, final=None):
    T, Fin = x.shape
    blk = 2000 if T == 100000 else 2048
    wq, wk, wv, wr = (_pad_w(p[n]["w"]) for n in ("q", "k", "v", "skip"))
    bc = jnp.concatenate(
        [_pad_vec(p[n]["b"]) for n in ("q", "k", "v", "skip")], axis=1)
    q, k, v, r = _tc_qkvr_kernel(T, Fin, blk)(x, wq, wk, wv, wr, bc)
    qd = _gather_rows(q, dst_g)
    ks = _gather_rows(k, src)
    vs = _gather_rows(v, src)
    Ep = src.shape[0]
    m = _tc_amax_kernel(Ep, eblk)(qd, ks)
    ws = _tc_exws_kernel(Ep, eblk)(qd, ks, vs, m)
    us = _scatter_rows(ws, dst_s, Tr, ranges)[:T]
    bw = p["beta_w"][:, 0]
    w1 = _pad_vec(bw[0:30])
    w2 = _pad_vec(bw[30:60])
    w3 = _pad_vec(bw[60:90])
    if final is None:
        return _tc_combine_kernel(T, blk, False)(us, r, w1, w2, w3)
    wf, bf = final
    return _tc_combine_kernel(T, blk, True)(us, r, w1, w2, w3, wf, bf)


@jax.jit
def _impl(graph_features, income, bonus_values_normed, batch, graph_edges,
          bonus_nodes, bonus_edges, bonus_batch, bonus_mapping, params):
    N = graph_features.shape[0]
    NB = bonus_nodes.shape[0]
    NBON = bonus_values_normed.shape[0]
    M = bonus_mapping.shape[1]
    E = graph_edges.shape[1]
    EB = bonus_edges.shape[1]
    Epad = 1638400
    EBpad = 819200
    NBpad = 212992
    BIG = 1 << 30
    p = params

    gf8 = jnp.pad(graph_features, ((0, 0), (0, 3)))
    inc8 = jnp.pad(income, ((0, 0), (0, 6)))
    wg = jnp.pad(p["init"]["w"][:5], ((0, 3), (0, 2)))
    wi = jnp.pad(p["init"]["w"][5:7], ((0, 6), (0, 2)))
    bi = _pad_vec(p["init"]["b"])
    bt2 = batch.astype(jnp.int32).reshape(N, 1)
    x = _tc_init_kernel(N, 2000)(gf8, inc8, wi, bt2, wg, bi)

    src = jnp.pad(graph_edges[0].astype(jnp.int32), (0, Epad - E))
    dst_g = jnp.pad(graph_edges[1].astype(jnp.int32), (0, Epad - E))
    dst_s = jnp.pad(graph_edges[1].astype(jnp.int32), (0, Epad - E),
                    constant_values=BIG)
    x = _conv(p["g1"], x, src, dst_g, dst_s, N, 1, 6400)
    x = _conv(p["g2"], x, src, dst_g, dst_s, N, 1, 6400)

    bn = jnp.pad(bonus_nodes.astype(jnp.int32), (0, NBpad - NB))
    xb = _gather_rows(x, bn)
    bsrc = jnp.pad(bonus_edges[0].astype(jnp.int32), (0, EBpad - EB))
    bdst_g = jnp.pad(bonus_edges[1].astype(jnp.int32), (0, EBpad - EB))
    bdst_s = jnp.pad(bonus_edges[1].astype(jnp.int32), (0, EBpad - EB),
                     constant_values=BIG)
    xb = _conv(p["b1"], xb, bsrc, bdst_g, bdst_s, NBpad // 2, 2, 6400)
    bb = jnp.pad(bonus_batch.astype(jnp.int32), (0, NBpad - NB),
                 constant_values=NBON)
    pooled = _scatter_rows(xb, bb, NBON, 1)
    c2 = _tc_scalemul_kernel(NBON, 2000)(
        pooled, bonus_values_normed.reshape(NBON, 1))
    cols = jnp.pad(bonus_mapping[1].astype(jnp.int32), (0, NBpad - M))
    rows = jnp.pad(bonus_mapping[0].astype(jnp.int32), (0, NBpad - M),
                   constant_values=BIG)
    gs = _gather_rows(c2, cols)
    bn20 = _scatter_rows(gs, rows, NBON, 1)
    bnode = jnp.concatenate(
        [bn20, jnp.zeros((N - NBON, F), jnp.float32)], axis=0)

    x3 = jnp.concatenate([x, bnode], axis=1)
    wf1 = _pad_w(p["final1"]["w"])
    bf1 = _pad_vec(p["final1"]["b"])
    x4 = _conv(p["g3"], x3, src, dst_g, dst_s, N, 1, 6400, final=(wf1, bf1))

    sums, counts = _tc_pool_kernel(N, 2000)(x4, bt2)
    w2v = _pad_vec(p["final2"]["w"][:, 0])
    b2s = p["final2"]["b"].reshape(1, 1)
    out = _tc_head_kernel()(sums, counts, w2v, b2s)
    return out.reshape(-1)


def kernel(graph_features, income, bonus_values_normed, batch, graph_edges,
           bonus_nodes, bonus_edges, bonus_batch, bonus_mapping, params):
    return _impl(graph_features, income, bonus_values_normed, batch,
                 graph_edges, bonus_nodes, bonus_edges, bonus_batch,
                 bonus_mapping, params)


# speedup vs baseline: 7.1757x; 1.0403x over previous
import functools
import math

import jax
import jax.numpy as jnp
from jax import lax
from jax.experimental import pallas as pl
from jax.experimental.pallas import tpu as pltpu
from jax.experimental.pallas import tpu_sc as plsc

F = 32
H = 16
NC = 2
NS = 16
NW = NC * NS
SCALE = 1.0 / math.sqrt(30.0)


def _pick_chunk(cnt):
    for c in range(128, 0, -8):
        if cnt % c == 0:
            return c
    raise ValueError(f"no chunk for {cnt}")


def _pick_block(cnt, maxbb=2048):
    for c in (2048, 1024, 512, 256, 128):
        if c <= maxbb and cnt % c == 0 and (cnt // c) % 2 == 0:
            return c
    raise ValueError(f"no block for {cnt}")


@functools.cache
def _gather_kernel(T, Ep, maxbb=1024):
    cnt = Ep // NW
    bb = _pick_block(cnt, maxbb=maxbb)
    nsub = bb // 128
    nb = cnt // bb
    nb2 = nb // 2
    mesh = plsc.VectorSubcoreMesh(core_axis_name="c", subcore_axis_name="s")

    def body(table, idx, out,
             idx_a, idx_b, rows_a, rows_b,
             sem_ia, sem_ib, sem_ga, sem_gb, sem_wa, sem_wb):
        wid = lax.axis_index("s") * NC + lax.axis_index("c")
        base = wid * cnt
        bufs = ((idx_a, rows_a, sem_ia, sem_ga, sem_wa),
                (idx_b, rows_b, sem_ib, sem_gb, sem_wb))

        def fire_idx(j, par):
            iv, _, si, _, _ = bufs[par]
            pltpu.async_copy(idx.at[pl.ds(base + j * bb, bb)], iv, si)

        def proc(j, par, first, last):
            iv, rv, si, sg, sw = bufs[par]
            if not last:
                @pl.when(j + 1 < nb)
                def _():
                    fire_idx(j + 1, 1 - par)
            pltpu.make_async_copy(idx.at[pl.ds(base, bb)], iv, si).wait()
            if not first:
                @pl.when(j >= 2)
                def _():
                    pltpu.make_async_copy(
                        rv, out.at[pl.ds(base, bb)], sw).wait()
            descs = [
                pltpu.async_copy(table.at[iv.at[pl.ds(k * 128, 128)]],
                                 rv.at[pl.ds(k * 128, 128)], sg)
                for k in range(nsub)
            ]
            for d in descs:
                d.wait()
            pltpu.async_copy(rv, out.at[pl.ds(base + j * bb, bb)], sw)

        fire_idx(0, 0)

        def step(jj, carry):
            proc(2 * jj, 0, False, False)
            proc(2 * jj + 1, 1, False, False)
            return carry

        lax.fori_loop(0, nb2, step, 0)
        for par in (0, 1):
            _, rv, _, _, sw = bufs[par]
            pltpu.make_async_copy(rv, out.at[pl.ds(base, bb)], sw).wait()

    return pl.kernel(
        body,
        out_type=jax.ShapeDtypeStruct((Ep, F), jnp.float32),
        mesh=mesh,
        compiler_params=pltpu.CompilerParams(use_tc_tiling_on_sc=False),
        scratch_types=[
            pltpu.VMEM((bb,), jnp.int32),
            pltpu.VMEM((bb,), jnp.int32),
            pltpu.VMEM((bb, F), jnp.float32),
            pltpu.VMEM((bb, F), jnp.float32),
        ] + [pltpu.SemaphoreType.DMA] * 6,
    )


def _gather_rows(table, idx):
    return _gather_kernel(table.shape[0], idx.shape[0])(table, idx)


@functools.cache
def _scatter_kernel(Ep, Tr, ranges):
    cnt = Ep // NS
    bb = _pick_block(cnt, maxbb=512)
    nsub = bb // 128
    nb = cnt // bb
    Tacc = Tr + 32
    wr = Tr // NS
    zr = Tacc // NS
    mesh = plsc.VectorSubcoreMesh(core_axis_name="c", subcore_axis_name="s")

    def body(vals, idx, zeros_hbm, out,
             idx_a, idx_b, midx_a, midx_b, vb_a, vb_b, acc,
             sem_ia, sem_ib, sem_va, sem_vb, sem_s):
        c = lax.axis_index("c")
        s = lax.axis_index("s")
        base = s * cnt
        bufs = ((idx_a, midx_a, vb_a, sem_ia, sem_va),
                (idx_b, midx_b, vb_b, sem_ib, sem_vb))

        def fire(j, par):
            iv, _, vv, si, sv = bufs[par]
            pltpu.async_copy(idx.at[pl.ds(base + j * bb, bb)], iv, si)
            pltpu.async_copy(
                vals.at[pl.ds(base + j * bb, bb), pl.ds(c * H, H)], vv, sv)

        for p in range(ranges):
            rbase = p * Tr
            pltpu.sync_copy(zeros_hbm.at[pl.ds(s * zr, zr)],
                            acc.at[pl.ds(s * zr, zr)])
            plsc.subcore_barrier()

            fire(0, 0)

            def proc(j, par):
                iv, mv, vv, si, sv = bufs[par]

                @pl.when(j + 1 < nb)
                def _():
                    fire(j + 1, 1 - par)

                pltpu.make_async_copy(
                    idx.at[pl.ds(base, bb)], iv, si).wait()
                for kk in range(bb // 16):
                    ivv = iv[pl.ds(kk * 16, 16)]
                    rel = ivv - rbase
                    ok = (rel >= 0) & (rel < Tr)
                    mv[kk // 8, pl.ds((kk % 8) * 16, 16)] = \
                        jnp.where(ok, rel, Tr)
                pltpu.make_async_copy(
                    vals.at[pl.ds(base, bb), pl.ds(c * H, H)], vv, sv).wait()
                descs = [
                    pltpu.async_copy(vv.at[pl.ds(k * 128, 128)],
                                     acc.at[mv.at[k]], sem_s, add=True)
                    for k in range(nsub)
                ]
                for d in descs:
                    d.wait()

            def step(jj, carry):
                proc(2 * jj, 0)
                proc(2 * jj + 1, 1)
                return carry

            lax.fori_loop(0, nb // 2, step, 0)
            plsc.subcore_barrier()
            pltpu.sync_copy(acc.at[pl.ds(s * wr, wr)],
                            out.at[pl.ds(rbase + s * wr, wr), pl.ds(c * H, H)])
            plsc.subcore_barrier()

    return pl.kernel(
        body,
        out_type=jax.ShapeDtypeStruct((Tr * ranges, F), jnp.float32),
        mesh=mesh,
        compiler_params=pltpu.CompilerParams(use_tc_tiling_on_sc=False),
        scratch_types=[
            pltpu.VMEM((bb,), jnp.int32),
            pltpu.VMEM((bb,), jnp.int32),
            pltpu.VMEM((nsub, 128), jnp.int32),
            pltpu.VMEM((nsub, 128), jnp.int32),
            pltpu.VMEM((bb, H), jnp.float32),
            pltpu.VMEM((bb, H), jnp.float32),
            pltpu.VMEM_SHARED((Tacc, H), jnp.float32),
        ] + [pltpu.SemaphoreType.DMA] * 5,
    )


def _scatter_rows(vals, idx, Tr, ranges):
    zeros_hbm = jnp.zeros((Tr + 32, H), jnp.float32)
    return _scatter_kernel(idx.shape[0], Tr, ranges)(vals, idx, zeros_hbm)


def _init_body(gf_ref, inc_ref, wi_ref, bt_ref, wg_ref, b_ref, o_ref):
    ip = jnp.dot(inc_ref[...], wi_ref[...], preferred_element_type=jnp.float32)
    oh = (bt_ref[...] == lax.broadcasted_iota(jnp.int32, (1, 64), 1)
          ).astype(jnp.float32)
    y = (jnp.dot(gf_ref[...], wg_ref[...], preferred_element_type=jnp.float32)
         + jnp.dot(oh, ip, preferred_element_type=jnp.float32) + b_ref[...])
    o_ref[...] = jnp.maximum(y, 0.0)


@functools.cache
def _tc_init_kernel(T, blk):
    grid = T // blk
    return pl.pallas_call(
        _init_body,
        grid=(grid,),
        in_specs=[
            pl.BlockSpec((blk, 8), lambda i: (i, 0)),
            pl.BlockSpec((64, 8), lambda i: (0, 0)),
            pl.BlockSpec((8, F), lambda i: (0, 0)),
            pl.BlockSpec((blk, 1), lambda i: (i, 0)),
            pl.BlockSpec((8, F), lambda i: (0, 0)),
            pl.BlockSpec((1, F), lambda i: (0, 0)),
        ],
        out_specs=pl.BlockSpec((blk, F), lambda i: (i, 0)),
        out_shape=jax.ShapeDtypeStruct((T, F), jnp.float32),
    )


def _qkvr_body(x_ref, wq, wk, wv, wr, b_ref, q_ref, k_ref, v_ref, r_ref):
    x = x_ref[...]
    b = b_ref[...]
    q_ref[...] = jnp.dot(x, wq[...], preferred_element_type=jnp.float32) + b[:, 0:F]
    k_ref[...] = jnp.dot(x, wk[...], preferred_element_type=jnp.float32) + b[:, F:2 * F]
    v_ref[...] = jnp.dot(x, wv[...], preferred_element_type=jnp.float32) + b[:, 2 * F:3 * F]
    r_ref[...] = jnp.dot(x, wr[...], preferred_element_type=jnp.float32) + b[:, 3 * F:4 * F]


@functools.cache
def _tc_qkvr_kernel(T, Fin, blk):
    grid = T // blk
    o = jax.ShapeDtypeStruct((T, F), jnp.float32)
    return pl.pallas_call(
        _qkvr_body,
        grid=(grid,),
        in_specs=[pl.BlockSpec((blk, Fin), lambda i: (i, 0))]
        + [pl.BlockSpec((Fin, F), lambda i: (0, 0))] * 4
        + [pl.BlockSpec((1, 4 * F), lambda i: (0, 0))],
        out_specs=[pl.BlockSpec((blk, F), lambda i: (i, 0))] * 4,
        out_shape=[o, o, o, o],
    )


def _amax_body(qd_ref, ks_ref, m_ref):
    i = pl.program_id(0)
    a = jnp.sum(qd_ref[...] * ks_ref[...], axis=1) * SCALE
    mx = jnp.max(a)

    @pl.when(i == 0)
    def _():
        m_ref[0, 0] = mx

    @pl.when(i > 0)
    def _():
        m_ref[0, 0] = jnp.maximum(m_ref[0, 0], mx)


@functools.cache
def _tc_amax_kernel(Ep, blk):
    grid = Ep // blk
    return pl.pallas_call(
        _amax_body,
        grid=(grid,),
        in_specs=[pl.BlockSpec((blk, F), lambda i: (i, 0))] * 2,
        out_specs=pl.BlockSpec(memory_space=pltpu.SMEM),
        out_shape=jax.ShapeDtypeStruct((1, 1), jnp.float32),
    )


def _exws_body(qd_ref, ks_ref, vs_ref, m_ref, w_ref):
    a = jnp.sum(qd_ref[...] * ks_ref[...], axis=1, keepdims=True) * SCALE
    ex = jnp.exp(a - m_ref[0, 0])
    col = lax.broadcasted_iota(jnp.int32, w_ref.shape, 1)
    w_ref[...] = vs_ref[...] * ex + jnp.where(col == F - 1, ex, 0.0)


@functools.cache
def _tc_exws_kernel(Ep, blk):
    grid = Ep // blk
    return pl.pallas_call(
        _exws_body,
        grid=(grid,),
        in_specs=[pl.BlockSpec((blk, F), lambda i: (i, 0))] * 3
        + [pl.BlockSpec(memory_space=pltpu.SMEM)],
        out_specs=pl.BlockSpec((blk, F), lambda i: (i, 0)),
        out_shape=jax.ShapeDtypeStruct((Ep, F), jnp.float32),
    )


def _combine_body(us_ref, r_ref, w1, w2, w3, o_ref):
    u = us_ref[...]
    s = u[:, F - 1:F]
    col = lax.broadcasted_iota(jnp.int32, u.shape, 1)
    out = jnp.where(col >= F - 2, 0.0, u / (s + 1e-16))
    r = r_ref[...]
    lg = jnp.sum(out * w1[...] + r * w2[...] + (out - r) * w3[...],
                 axis=1, keepdims=True)
    beta = 1.0 / (1.0 + jnp.exp(-lg))
    o_ref[...] = jnp.maximum(beta * r + (1.0 - beta) * out, 0.0)


def _combine_final_body(us_ref, r_ref, w1, w2, w3, wf, bf, o_ref):
    u = us_ref[...]
    s = u[:, F - 1:F]
    col = lax.broadcasted_iota(jnp.int32, u.shape, 1)
    out = jnp.where(col >= F - 2, 0.0, u / (s + 1e-16))
    r = r_ref[...]
    lg = jnp.sum(out * w1[...] + r * w2[...] + (out - r) * w3[...],
                 axis=1, keepdims=True)
    beta = 1.0 / (1.0 + jnp.exp(-lg))
    x = jnp.maximum(beta * r + (1.0 - beta) * out, 0.0)
    y = jnp.dot(x, wf[...], preferred_element_type=jnp.float32) + bf[...]
    o_ref[...] = jnp.maximum(y, 0.0)


@functools.cache
def _tc_combine_kernel(T, blk, with_final):
    grid = T // blk
    specs = [pl.BlockSpec((blk, F), lambda i: (i, 0))] * 2 \
        + [pl.BlockSpec((1, F), lambda i: (0, 0))] * 3
    body = _combine_body
    if with_final:
        specs += [pl.BlockSpec((F, F), lambda i: (0, 0)),
                  pl.BlockSpec((1, F), lambda i: (0, 0))]
        body = _combine_final_body
    return pl.pallas_call(
        body,
        grid=(grid,),
        in_specs=specs,
        out_specs=pl.BlockSpec((blk, F), lambda i: (i, 0)),
        out_shape=jax.ShapeDtypeStruct((T, F), jnp.float32),
    )


def _scalemul_body(b_ref, v_ref, o_ref):
    o_ref[...] = b_ref[...] * v_ref[...]


@functools.cache
def _tc_scalemul_kernel(T, blk):
    return pl.pallas_call(
        _scalemul_body,
        grid=(T // blk,),
        in_specs=[pl.BlockSpec((blk, F), lambda i: (i, 0)),
                  pl.BlockSpec((blk, 1), lambda i: (i, 0))],
        out_specs=pl.BlockSpec((blk, F), lambda i: (i, 0)),
        out_shape=jax.ShapeDtypeStruct((T, F), jnp.float32),
    )


def _pool_body(x_ref, bt_ref, s_ref, c_ref):
    i = pl.program_id(0)

    @pl.when(i == 0)
    def _():
        s_ref[...] = jnp.zeros_like(s_ref)
        c_ref[...] = jnp.zeros_like(c_ref)

    oh = (bt_ref[...] == lax.broadcasted_iota(jnp.int32, (1, 64), 1)
          ).astype(jnp.float32)
    s_ref[...] += lax.dot_general(oh, x_ref[...], (((0,), (0,)), ((), ())),
                                  preferred_element_type=jnp.float32)
    c_ref[...] += lax.dot_general(oh, jnp.ones_like(bt_ref[...], jnp.float32),
                                  (((0,), (0,)), ((), ())),
                                  preferred_element_type=jnp.float32)


@functools.cache
def _tc_pool_kernel(T, blk):
    return pl.pallas_call(
        _pool_body,
        grid=(T // blk,),
        in_specs=[pl.BlockSpec((blk, F), lambda i: (i, 0)),
                  pl.BlockSpec((blk, 1), lambda i: (i, 0))],
        out_specs=[pl.BlockSpec((64, F), lambda i: (0, 0)),
                   pl.BlockSpec((64, 1), lambda i: (0, 0))],
        out_shape=[jax.ShapeDtypeStruct((64, F), jnp.float32),
                   jax.ShapeDtypeStruct((64, 1), jnp.float32)],
    )


def _head_body(s_ref, c_ref, w_ref, b_ref, o_ref):
    m = s_ref[...] / jnp.maximum(c_ref[...], 1.0)
    v = jnp.sum(m * w_ref[...], axis=1, keepdims=True) + b_ref[0, 0]
    o_ref[...] = jnp.tanh(v)


@functools.cache
def _tc_head_kernel():
    return pl.pallas_call(
        _head_body,
        grid=(1,),
        in_specs=[pl.BlockSpec((64, F), lambda i: (0, 0)),
                  pl.BlockSpec((64, 1), lambda i: (0, 0)),
                  pl.BlockSpec((1, F), lambda i: (0, 0)),
                  pl.BlockSpec((1, 1), lambda i: (0, 0))],
        out_specs=pl.BlockSpec((64, 1), lambda i: (0, 0)),
        out_shape=jax.ShapeDtypeStruct((64, 1), jnp.float32),
    )


def _pad_w(w):
    din, dout = w.shape
    if din > 32:
        h = din // 2
        top = jnp.pad(w[:h], ((0, 32 - h), (0, F - dout)))
        bot = jnp.pad(w[h:], ((0, 32 - (din - h)), (0, F - dout)))
        return jnp.concatenate([top, bot], axis=0)
    return jnp.pad(w, ((0, 32 - din), (0, F - dout)))


def _pad_vec(b):
    return jnp.pad(b, (0, F - b.shape[0])).reshape(1, F)


def _conv(p, x, src, dst_g, dst_s, Tr, ranges, eblk, final=None):
    T, Fin = x.shape
    blk = 2000 if T == 100000 else 2048
    wq, wk, wv, wr = (_pad_w(p[n]["w"]) for n in ("q", "k", "v", "skip"))
    bc = jnp.concatenate(
        [_pad_vec(p[n]["b"]) for n in ("q", "k", "v", "skip")], axis=1)
    q, k, v, r = _tc_qkvr_kernel(T, Fin, blk)(x, wq, wk, wv, wr, bc)
    qd = _gather_rows(q, dst_g)
    ks = _gather_rows(k, src)
    vs = _gather_rows(v, src)
    Ep = src.shape[0]
    m = _tc_amax_kernel(Ep, eblk)(qd, ks)
    ws = _tc_exws_kernel(Ep, eblk)(qd, ks, vs, m)
    us = _scatter_rows(ws, dst_s, Tr, ranges)[:T]
    bw = p["beta_w"][:, 0]
    w1 = _pad_vec(bw[0:30])
    w2 = _pad_vec(bw[30:60])
    w3 = _pad_vec(bw[60:90])
    if final is None:
        return _tc_combine_kernel(T, blk, False)(us, r, w1, w2, w3)
    wf, bf = final
    return _tc_combine_kernel(T, blk, True)(us, r, w1, w2, w3, wf, bf)


@jax.jit
def _impl(graph_features, income, bonus_values_normed, batch, graph_edges,
          bonus_nodes, bonus_edges, bonus_batch, bonus_mapping, params):
    N = graph_features.shape[0]
    NB = bonus_nodes.shape[0]
    NBON = bonus_values_normed.shape[0]
    M = bonus_mapping.shape[1]
    E = graph_edges.shape[1]
    EB = bonus_edges.shape[1]
    Epad = 1638400
    EBpad = 819200
    NBpad = 212992
    BIG = 1 << 30
    p = params

    gf8 = jnp.pad(graph_features, ((0, 0), (0, 3)))
    inc8 = jnp.pad(income, ((0, 0), (0, 6)))
    wg = jnp.pad(p["init"]["w"][:5], ((0, 3), (0, 2)))
    wi = jnp.pad(p["init"]["w"][5:7], ((0, 6), (0, 2)))
    bi = _pad_vec(p["init"]["b"])
    bt2 = batch.astype(jnp.int32).reshape(N, 1)
    x = _tc_init_kernel(N, 2000)(gf8, inc8, wi, bt2, wg, bi)

    src = jnp.pad(graph_edges[0].astype(jnp.int32), (0, Epad - E))
    dst_g = jnp.pad(graph_edges[1].astype(jnp.int32), (0, Epad - E))
    dst_s = jnp.pad(graph_edges[1].astype(jnp.int32), (0, Epad - E),
                    constant_values=BIG)
    x = _conv(p["g1"], x, src, dst_g, dst_s, N, 1, 6400)
    x = _conv(p["g2"], x, src, dst_g, dst_s, N, 1, 6400)

    bn = jnp.pad(bonus_nodes.astype(jnp.int32), (0, NBpad - NB))
    xb = _gather_rows(x, bn)
    bsrc = jnp.pad(bonus_edges[0].astype(jnp.int32), (0, EBpad - EB))
    bdst_g = jnp.pad(bonus_edges[1].astype(jnp.int32), (0, EBpad - EB))
    bdst_s = jnp.pad(bonus_edges[1].astype(jnp.int32), (0, EBpad - EB),
                     constant_values=BIG)
    xb = _conv(p["b1"], xb, bsrc, bdst_g, bdst_s, NBpad // 2, 2, 6400)
    bb = jnp.pad(bonus_batch.astype(jnp.int32), (0, NBpad - NB),
                 constant_values=NBON)
    pooled = _scatter_rows(xb, bb, NBON, 1)
    c2 = _tc_scalemul_kernel(NBON, 2000)(
        pooled, bonus_values_normed.reshape(NBON, 1))
    cols = jnp.pad(bonus_mapping[1].astype(jnp.int32), (0, NBpad - M))
    rows = jnp.pad(bonus_mapping[0].astype(jnp.int32), (0, NBpad - M),
                   constant_values=BIG)
    gs = _gather_rows(c2, cols)
    bn20 = _scatter_rows(gs, rows, NBON, 1)
    bnode = jnp.concatenate(
        [bn20, jnp.zeros((N - NBON, F), jnp.float32)], axis=0)

    x3 = jnp.concatenate([x, bnode], axis=1)
    wf1 = _pad_w(p["final1"]["w"])
    bf1 = _pad_vec(p["final1"]["b"])
    x4 = _conv(p["g3"], x3, src, dst_g, dst_s, N, 1, 6400, final=(wf1, bf1))

    sums, counts = _tc_pool_kernel(N, 2000)(x4, bt2)
    w2v = _pad_vec(p["final2"]["w"][:, 0])
    b2s = p["final2"]["b"].reshape(1, 1)
    out = _tc_head_kernel()(sums, counts, w2v, b2s)
    return out.reshape(-1)


def kernel(graph_features, income, bonus_values_normed, batch, graph_edges,
           bonus_nodes, bonus_edges, bonus_batch, bonus_mapping, params):
    return _impl(graph_features, income, bonus_values_normed, batch,
                 graph_edges, bonus_nodes, bonus_edges, bonus_batch,
                 bonus_mapping, params)


# speedup vs baseline: 7.6722x; 1.0692x over previous
import functools
import math

import jax
import jax.numpy as jnp
from jax import lax
from jax.experimental import pallas as pl
from jax.experimental.pallas import tpu as pltpu
from jax.experimental.pallas import tpu_sc as plsc

F = 32
H = 16
NC = 2
NS = 16
NW = NC * NS
SCALE = 1.0 / math.sqrt(30.0)


def _pick_chunk(cnt):
    for c in range(128, 0, -8):
        if cnt % c == 0:
            return c
    raise ValueError(f"no chunk for {cnt}")


def _pick_block(cnt, maxbb=2048):
    for c in (2048, 1024, 512, 256, 128):
        if c <= maxbb and cnt % c == 0 and (cnt // c) % 2 == 0:
            return c
    raise ValueError(f"no block for {cnt}")


@functools.cache
def _gather_kernel(T, Ep, D=F, maxbb=1024):
    cnt = Ep // NW
    bb = _pick_block(cnt, maxbb=maxbb)
    sub = min(512, bb)
    nsub = bb // sub
    nb = cnt // bb
    nb2 = nb // 2
    mesh = plsc.VectorSubcoreMesh(core_axis_name="c", subcore_axis_name="s")

    def body(table, idx, out,
             idx_a, idx_b, rows_a, rows_b,
             sem_ia, sem_ib, sem_ga, sem_gb, sem_wa, sem_wb):
        wid = lax.axis_index("s") * NC + lax.axis_index("c")
        base = wid * cnt
        bufs = ((idx_a, rows_a, sem_ia, sem_ga, sem_wa),
                (idx_b, rows_b, sem_ib, sem_gb, sem_wb))

        def fire_idx(j, par):
            iv, _, si, _, _ = bufs[par]
            pltpu.async_copy(idx.at[pl.ds(base + j * bb, bb)], iv, si)

        def proc(j, par, first, last):
            iv, rv, si, sg, sw = bufs[par]
            if not last:
                @pl.when(j + 1 < nb)
                def _():
                    fire_idx(j + 1, 1 - par)
            pltpu.make_async_copy(idx.at[pl.ds(base, bb)], iv, si).wait()
            if not first:
                @pl.when(j >= 2)
                def _():
                    pltpu.make_async_copy(
                        rv, out.at[pl.ds(base, bb)], sw).wait()
            descs = [
                pltpu.async_copy(table.at[iv.at[pl.ds(k * sub, sub)]],
                                 rv.at[pl.ds(k * sub, sub)], sg)
                for k in range(nsub)
            ]
            for d in descs:
                d.wait()
            pltpu.async_copy(rv, out.at[pl.ds(base + j * bb, bb)], sw)

        fire_idx(0, 0)

        def step(jj, carry):
            proc(2 * jj, 0, False, False)
            proc(2 * jj + 1, 1, False, False)
            return carry

        lax.fori_loop(0, nb2, step, 0)
        for par in (0, 1):
            _, rv, _, _, sw = bufs[par]
            pltpu.make_async_copy(rv, out.at[pl.ds(base, bb)], sw).wait()

    return pl.kernel(
        body,
        out_type=jax.ShapeDtypeStruct((Ep, D), jnp.float32),
        mesh=mesh,
        compiler_params=pltpu.CompilerParams(use_tc_tiling_on_sc=False),
        scratch_types=[
            pltpu.VMEM((bb,), jnp.int32),
            pltpu.VMEM((bb,), jnp.int32),
            pltpu.VMEM((bb, D), jnp.float32),
            pltpu.VMEM((bb, D), jnp.float32),
        ] + [pltpu.SemaphoreType.DMA] * 6,
    )


def _gather_rows(table, idx):
    D = table.shape[1]
    maxbb = 1024 if D <= 32 else 512
    return _gather_kernel(table.shape[0], idx.shape[0], D, maxbb)(table, idx)


@functools.cache
def _scatter_kernel(Ep, Tr, ranges):
    cnt = Ep // NS
    bb = _pick_block(cnt, maxbb=512)
    nsub = bb // 128
    nb = cnt // bb
    Tacc = Tr + 32
    wr = Tr // NS
    zr = Tacc // NS
    mesh = plsc.VectorSubcoreMesh(core_axis_name="c", subcore_axis_name="s")

    def body(vals, idx, zeros_hbm, out,
             idx_a, idx_b, midx_a, midx_b, vb_a, vb_b, acc,
             sem_ia, sem_ib, sem_va, sem_vb, sem_s):
        c = lax.axis_index("c")
        s = lax.axis_index("s")
        base = s * cnt
        bufs = ((idx_a, midx_a, vb_a, sem_ia, sem_va),
                (idx_b, midx_b, vb_b, sem_ib, sem_vb))

        def fire(j, par):
            iv, _, vv, si, sv = bufs[par]
            pltpu.async_copy(idx.at[pl.ds(base + j * bb, bb)], iv, si)
            pltpu.async_copy(
                vals.at[pl.ds(base + j * bb, bb), pl.ds(c * H, H)], vv, sv)

        for p in range(ranges):
            rbase = p * Tr
            pltpu.sync_copy(zeros_hbm.at[pl.ds(s * zr, zr)],
                            acc.at[pl.ds(s * zr, zr)])
            plsc.subcore_barrier()

            fire(0, 0)

            def proc(j, par):
                iv, mv, vv, si, sv = bufs[par]

                @pl.when(j + 1 < nb)
                def _():
                    fire(j + 1, 1 - par)

                pltpu.make_async_copy(
                    idx.at[pl.ds(base, bb)], iv, si).wait()
                for kk in range(bb // 16):
                    ivv = iv[pl.ds(kk * 16, 16)]
                    rel = ivv - rbase
                    ok = (rel >= 0) & (rel < Tr)
                    mv[kk // 8, pl.ds((kk % 8) * 16, 16)] = \
                        jnp.where(ok, rel, Tr)
                pltpu.make_async_copy(
                    vals.at[pl.ds(base, bb), pl.ds(c * H, H)], vv, sv).wait()
                descs = [
                    pltpu.async_copy(vv.at[pl.ds(k * 128, 128)],
                                     acc.at[mv.at[k]], sem_s, add=True)
                    for k in range(nsub)
                ]
                for d in descs:
                    d.wait()

            def step(jj, carry):
                proc(2 * jj, 0)
                proc(2 * jj + 1, 1)
                return carry

            lax.fori_loop(0, nb // 2, step, 0)
            plsc.subcore_barrier()
            pltpu.sync_copy(acc.at[pl.ds(s * wr, wr)],
                            out.at[pl.ds(rbase + s * wr, wr), pl.ds(c * H, H)])
            plsc.subcore_barrier()

    return pl.kernel(
        body,
        out_type=jax.ShapeDtypeStruct((Tr * ranges, F), jnp.float32),
        mesh=mesh,
        compiler_params=pltpu.CompilerParams(use_tc_tiling_on_sc=False),
        scratch_types=[
            pltpu.VMEM((bb,), jnp.int32),
            pltpu.VMEM((bb,), jnp.int32),
            pltpu.VMEM((nsub, 128), jnp.int32),
            pltpu.VMEM((nsub, 128), jnp.int32),
            pltpu.VMEM((bb, H), jnp.float32),
            pltpu.VMEM((bb, H), jnp.float32),
            pltpu.VMEM_SHARED((Tacc, H), jnp.float32),
        ] + [pltpu.SemaphoreType.DMA] * 5,
    )


def _scatter_rows(vals, idx, Tr, ranges):
    zeros_hbm = jnp.zeros((Tr + 32, H), jnp.float32)
    return _scatter_kernel(idx.shape[0], Tr, ranges)(vals, idx, zeros_hbm)


def _init_body(gf_ref, inc_ref, wi_ref, bt_ref, wg_ref, b_ref, o_ref):
    ip = jnp.dot(inc_ref[...], wi_ref[...], preferred_element_type=jnp.float32)
    oh = (bt_ref[...] == lax.broadcasted_iota(jnp.int32, (1, 64), 1)
          ).astype(jnp.float32)
    y = (jnp.dot(gf_ref[...], wg_ref[...], preferred_element_type=jnp.float32)
         + jnp.dot(oh, ip, preferred_element_type=jnp.float32) + b_ref[...])
    o_ref[...] = jnp.maximum(y, 0.0)


@functools.cache
def _tc_init_kernel(T, blk):
    grid = T // blk
    return pl.pallas_call(
        _init_body,
        grid=(grid,),
        in_specs=[
            pl.BlockSpec((blk, 8), lambda i: (i, 0)),
            pl.BlockSpec((64, 8), lambda i: (0, 0)),
            pl.BlockSpec((8, F), lambda i: (0, 0)),
            pl.BlockSpec((blk, 1), lambda i: (i, 0)),
            pl.BlockSpec((8, F), lambda i: (0, 0)),
            pl.BlockSpec((1, F), lambda i: (0, 0)),
        ],
        out_specs=pl.BlockSpec((blk, F), lambda i: (i, 0)),
        out_shape=jax.ShapeDtypeStruct((T, F), jnp.float32),
    )


def _qkvr_body(x_ref, wq, wkv, wr, b_ref, q_ref, kv_ref, r_ref):
    x = x_ref[...]
    b = b_ref[...]
    q_ref[...] = jnp.dot(x, wq[...], preferred_element_type=jnp.float32) + b[:, 0:F]
    kv_ref[...] = jnp.dot(x, wkv[...], preferred_element_type=jnp.float32) + b[:, F:3 * F]
    r_ref[...] = jnp.dot(x, wr[...], preferred_element_type=jnp.float32) + b[:, 3 * F:4 * F]


@functools.cache
def _tc_qkvr_kernel(T, Fin, blk):
    grid = T // blk
    return pl.pallas_call(
        _qkvr_body,
        grid=(grid,),
        in_specs=[pl.BlockSpec((blk, Fin), lambda i: (i, 0)),
                  pl.BlockSpec((Fin, F), lambda i: (0, 0)),
                  pl.BlockSpec((Fin, 2 * F), lambda i: (0, 0)),
                  pl.BlockSpec((Fin, F), lambda i: (0, 0)),
                  pl.BlockSpec((1, 4 * F), lambda i: (0, 0))],
        out_specs=[pl.BlockSpec((blk, F), lambda i: (i, 0)),
                   pl.BlockSpec((blk, 2 * F), lambda i: (i, 0)),
                   pl.BlockSpec((blk, F), lambda i: (i, 0))],
        out_shape=[jax.ShapeDtypeStruct((T, F), jnp.float32),
                   jax.ShapeDtypeStruct((T, 2 * F), jnp.float32),
                   jax.ShapeDtypeStruct((T, F), jnp.float32)],
    )


def _amax_body(qd_ref, kv_ref, m_ref):
    i = pl.program_id(0)
    a = jnp.sum(qd_ref[...] * kv_ref[:, 0:F], axis=1) * SCALE
    mx = jnp.max(a)

    @pl.when(i == 0)
    def _():
        m_ref[0, 0] = mx

    @pl.when(i > 0)
    def _():
        m_ref[0, 0] = jnp.maximum(m_ref[0, 0], mx)


@functools.cache
def _tc_amax_kernel(Ep, blk):
    grid = Ep // blk
    return pl.pallas_call(
        _amax_body,
        grid=(grid,),
        in_specs=[pl.BlockSpec((blk, F), lambda i: (i, 0)),
                  pl.BlockSpec((blk, 2 * F), lambda i: (i, 0))],
        out_specs=pl.BlockSpec(memory_space=pltpu.SMEM),
        out_shape=jax.ShapeDtypeStruct((1, 1), jnp.float32),
    )


def _exws_body(qd_ref, kv_ref, m_ref, w_ref):
    kv = kv_ref[...]
    a = jnp.sum(qd_ref[...] * kv[:, 0:F], axis=1, keepdims=True) * SCALE
    ex = jnp.exp(a - m_ref[0, 0])
    col = lax.broadcasted_iota(jnp.int32, w_ref.shape, 1)
    w_ref[...] = kv[:, F:2 * F] * ex + jnp.where(col == F - 1, ex, 0.0)


@functools.cache
def _tc_exws_kernel(Ep, blk):
    grid = Ep // blk
    return pl.pallas_call(
        _exws_body,
        grid=(grid,),
        in_specs=[pl.BlockSpec((blk, F), lambda i: (i, 0)),
                  pl.BlockSpec((blk, 2 * F), lambda i: (i, 0)),
                  pl.BlockSpec(memory_space=pltpu.SMEM)],
        out_specs=pl.BlockSpec((blk, F), lambda i: (i, 0)),
        out_shape=jax.ShapeDtypeStruct((Ep, F), jnp.float32),
    )


def _combine_body(us_ref, r_ref, w1, w2, w3, o_ref):
    u = us_ref[...]
    s = u[:, F - 1:F]
    col = lax.broadcasted_iota(jnp.int32, u.shape, 1)
    out = jnp.where(col >= F - 2, 0.0, u / (s + 1e-16))
    r = r_ref[...]
    lg = jnp.sum(out * w1[...] + r * w2[...] + (out - r) * w3[...],
                 axis=1, keepdims=True)
    beta = 1.0 / (1.0 + jnp.exp(-lg))
    o_ref[...] = jnp.maximum(beta * r + (1.0 - beta) * out, 0.0)


def _combine_final_body(us_ref, r_ref, w1, w2, w3, wf, bf, o_ref):
    u = us_ref[...]
    s = u[:, F - 1:F]
    col = lax.broadcasted_iota(jnp.int32, u.shape, 1)
    out = jnp.where(col >= F - 2, 0.0, u / (s + 1e-16))
    r = r_ref[...]
    lg = jnp.sum(out * w1[...] + r * w2[...] + (out - r) * w3[...],
                 axis=1, keepdims=True)
    beta = 1.0 / (1.0 + jnp.exp(-lg))
    x = jnp.maximum(beta * r + (1.0 - beta) * out, 0.0)
    y = jnp.dot(x, wf[...], preferred_element_type=jnp.float32) + bf[...]
    o_ref[...] = jnp.maximum(y, 0.0)


@functools.cache
def _tc_combine_kernel(T, blk, with_final):
    grid = T // blk
    specs = [pl.BlockSpec((blk, F), lambda i: (i, 0))] * 2 \
        + [pl.BlockSpec((1, F), lambda i: (0, 0))] * 3
    body = _combine_body
    if with_final:
        specs += [pl.BlockSpec((F, F), lambda i: (0, 0)),
                  pl.BlockSpec((1, F), lambda i: (0, 0))]
        body = _combine_final_body
    return pl.pallas_call(
        body,
        grid=(grid,),
        in_specs=specs,
        out_specs=pl.BlockSpec((blk, F), lambda i: (i, 0)),
        out_shape=jax.ShapeDtypeStruct((T, F), jnp.float32),
    )


def _scalemul_body(b_ref, v_ref, o_ref):
    o_ref[...] = b_ref[...] * v_ref[...]


@functools.cache
def _tc_scalemul_kernel(T, blk):
    return pl.pallas_call(
        _scalemul_body,
        grid=(T // blk,),
        in_specs=[pl.BlockSpec((blk, F), lambda i: (i, 0)),
                  pl.BlockSpec((blk, 1), lambda i: (i, 0))],
        out_specs=pl.BlockSpec((blk, F), lambda i: (i, 0)),
        out_shape=jax.ShapeDtypeStruct((T, F), jnp.float32),
    )


def _pool_body(x_ref, bt_ref, s_ref, c_ref):
    i = pl.program_id(0)

    @pl.when(i == 0)
    def _():
        s_ref[...] = jnp.zeros_like(s_ref)
        c_ref[...] = jnp.zeros_like(c_ref)

    oh = (bt_ref[...] == lax.broadcasted_iota(jnp.int32, (1, 64), 1)
          ).astype(jnp.float32)
    s_ref[...] += lax.dot_general(oh, x_ref[...], (((0,), (0,)), ((), ())),
                                  preferred_element_type=jnp.float32)
    c_ref[...] += lax.dot_general(oh, jnp.ones_like(bt_ref[...], jnp.float32),
                                  (((0,), (0,)), ((), ())),
                                  preferred_element_type=jnp.float32)


@functools.cache
def _tc_pool_kernel(T, blk):
    return pl.pallas_call(
        _pool_body,
        grid=(T // blk,),
        in_specs=[pl.BlockSpec((blk, F), lambda i: (i, 0)),
                  pl.BlockSpec((blk, 1), lambda i: (i, 0))],
        out_specs=[pl.BlockSpec((64, F), lambda i: (0, 0)),
                   pl.BlockSpec((64, 1), lambda i: (0, 0))],
        out_shape=[jax.ShapeDtypeStruct((64, F), jnp.float32),
                   jax.ShapeDtypeStruct((64, 1), jnp.float32)],
    )


def _head_body(s_ref, c_ref, w_ref, b_ref, o_ref):
    m = s_ref[...] / jnp.maximum(c_ref[...], 1.0)
    v = jnp.sum(m * w_ref[...], axis=1, keepdims=True) + b_ref[0, 0]
    o_ref[...] = jnp.tanh(v)


@functools.cache
def _tc_head_kernel():
    return pl.pallas_call(
        _head_body,
        grid=(1,),
        in_specs=[pl.BlockSpec((64, F), lambda i: (0, 0)),
                  pl.BlockSpec((64, 1), lambda i: (0, 0)),
                  pl.BlockSpec((1, F), lambda i: (0, 0)),
                  pl.BlockSpec((1, 1), lambda i: (0, 0))],
        out_specs=pl.BlockSpec((64, 1), lambda i: (0, 0)),
        out_shape=jax.ShapeDtypeStruct((64, 1), jnp.float32),
    )


def _pad_w(w):
    din, dout = w.shape
    if din > 32:
        h = din // 2
        top = jnp.pad(w[:h], ((0, 32 - h), (0, F - dout)))
        bot = jnp.pad(w[h:], ((0, 32 - (din - h)), (0, F - dout)))
        return jnp.concatenate([top, bot], axis=0)
    return jnp.pad(w, ((0, 32 - din), (0, F - dout)))


def _pad_vec(b):
    return jnp.pad(b, (0, F - b.shape[0])).reshape(1, F)


def _conv(p, x, src, dst_g, dst_s, Tr, ranges, eblk, final=None):
    T, Fin = x.shape
    blk = 2000 if T == 100000 else 2048
    wq, wk, wv, wr = (_pad_w(p[n]["w"]) for n in ("q", "k", "v", "skip"))
    wkv = jnp.concatenate([wk, wv], axis=1)
    bc = jnp.concatenate(
        [_pad_vec(p[n]["b"]) for n in ("q", "k", "v", "skip")], axis=1)
    q, kv, r = _tc_qkvr_kernel(T, Fin, blk)(x, wq, wkv, wr, bc)
    qd = _gather_rows(q, dst_g)
    kvs = _gather_rows(kv, src)
    Ep = src.shape[0]
    m = _tc_amax_kernel(Ep, eblk)(qd, kvs)
    ws = _tc_exws_kernel(Ep, eblk)(qd, kvs, m)
    us = _scatter_rows(ws, dst_s, Tr, ranges)[:T]
    bw = p["beta_w"][:, 0]
    w1 = _pad_vec(bw[0:30])
    w2 = _pad_vec(bw[30:60])
    w3 = _pad_vec(bw[60:90])
    if final is None:
        return _tc_combine_kernel(T, blk, False)(us, r, w1, w2, w3)
    wf, bf = final
    return _tc_combine_kernel(T, blk, True)(us, r, w1, w2, w3, wf, bf)


@jax.jit
def _impl(graph_features, income, bonus_values_normed, batch, graph_edges,
          bonus_nodes, bonus_edges, bonus_batch, bonus_mapping, params):
    N = graph_features.shape[0]
    NB = bonus_nodes.shape[0]
    NBON = bonus_values_normed.shape[0]
    M = bonus_mapping.shape[1]
    E = graph_edges.shape[1]
    EB = bonus_edges.shape[1]
    Epad = 1638400
    EBpad = 819200
    NBpad = 212992
    BIG = 1 << 30
    p = params

    gf8 = jnp.pad(graph_features, ((0, 0), (0, 3)))
    inc8 = jnp.pad(income, ((0, 0), (0, 6)))
    wg = jnp.pad(p["init"]["w"][:5], ((0, 3), (0, 2)))
    wi = jnp.pad(p["init"]["w"][5:7], ((0, 6), (0, 2)))
    bi = _pad_vec(p["init"]["b"])
    bt2 = batch.astype(jnp.int32).reshape(N, 1)
    x = _tc_init_kernel(N, 2000)(gf8, inc8, wi, bt2, wg, bi)

    src = jnp.pad(graph_edges[0].astype(jnp.int32), (0, Epad - E))
    dst_g = jnp.pad(graph_edges[1].astype(jnp.int32), (0, Epad - E))
    dst_s = jnp.pad(graph_edges[1].astype(jnp.int32), (0, Epad - E),
                    constant_values=BIG)
    x = _conv(p["g1"], x, src, dst_g, dst_s, N, 1, 6400)
    x = _conv(p["g2"], x, src, dst_g, dst_s, N, 1, 6400)

    bn = jnp.pad(bonus_nodes.astype(jnp.int32), (0, NBpad - NB))
    xb = _gather_rows(x, bn)
    bsrc = jnp.pad(bonus_edges[0].astype(jnp.int32), (0, EBpad - EB))
    bdst_g = jnp.pad(bonus_edges[1].astype(jnp.int32), (0, EBpad - EB))
    bdst_s = jnp.pad(bonus_edges[1].astype(jnp.int32), (0, EBpad - EB),
                     constant_values=BIG)
    xb = _conv(p["b1"], xb, bsrc, bdst_g, bdst_s, NBpad // 2, 2, 6400)
    bb = jnp.pad(bonus_batch.astype(jnp.int32), (0, NBpad - NB),
                 constant_values=NBON)
    pooled = _scatter_rows(xb, bb, NBON, 1)
    c2 = _tc_scalemul_kernel(NBON, 2000)(
        pooled, bonus_values_normed.reshape(NBON, 1))
    cols = jnp.pad(bonus_mapping[1].astype(jnp.int32), (0, NBpad - M))
    rows = jnp.pad(bonus_mapping[0].astype(jnp.int32), (0, NBpad - M),
                   constant_values=BIG)
    gs = _gather_rows(c2, cols)
    bn20 = _scatter_rows(gs, rows, NBON, 1)
    bnode = jnp.concatenate(
        [bn20, jnp.zeros((N - NBON, F), jnp.float32)], axis=0)

    x3 = jnp.concatenate([x, bnode], axis=1)
    wf1 = _pad_w(p["final1"]["w"])
    bf1 = _pad_vec(p["final1"]["b"])
    x4 = _conv(p["g3"], x3, src, dst_g, dst_s, N, 1, 6400, final=(wf1, bf1))

    sums, counts = _tc_pool_kernel(N, 2000)(x4, bt2)
    w2v = _pad_vec(p["final2"]["w"][:, 0])
    b2s = p["final2"]["b"].reshape(1, 1)
    out = _tc_head_kernel()(sums, counts, w2v, b2s)
    return out.reshape(-1)


def kernel(graph_features, income, bonus_values_normed, batch, graph_edges,
           bonus_nodes, bonus_edges, bonus_batch, bonus_mapping, params):
    return _impl(graph_features, income, bonus_values_normed, batch,
                 graph_edges, bonus_nodes, bonus_edges, bonus_batch,
                 bonus_mapping, params)


# speedup vs baseline: 8.2117x; 1.0703x over previous
import functools
import math

import jax
import jax.numpy as jnp
from jax import lax
from jax.experimental import pallas as pl
from jax.experimental.pallas import tpu as pltpu
from jax.experimental.pallas import tpu_sc as plsc

F = 32
H = 16
NC = 2
NS = 16
NW = NC * NS
SCALE = 1.0 / math.sqrt(30.0)


def _pick_chunk(cnt):
    for c in range(128, 0, -8):
        if cnt % c == 0:
            return c
    raise ValueError(f"no chunk for {cnt}")


def _pick_block(cnt, maxbb=2048):
    for c in (2048, 1024, 512, 256, 128):
        if c <= maxbb and cnt % c == 0 and (cnt // c) % 2 == 0:
            return c
    raise ValueError(f"no block for {cnt}")


@functools.cache
def _gather_kernel(T, Ep, D=F, maxbb=1024):
    cnt = Ep // NW
    bb = _pick_block(cnt, maxbb=maxbb)
    sub = min(512, bb)
    nsub = bb // sub
    nb = cnt // bb
    nb2 = nb // 2
    mesh = plsc.VectorSubcoreMesh(core_axis_name="c", subcore_axis_name="s")

    def body(table, idx, out,
             idx_a, idx_b, rows_a, rows_b,
             sem_ia, sem_ib, sem_ga, sem_gb, sem_wa, sem_wb):
        wid = lax.axis_index("s") * NC + lax.axis_index("c")
        base = wid * cnt
        bufs = ((idx_a, rows_a, sem_ia, sem_ga, sem_wa),
                (idx_b, rows_b, sem_ib, sem_gb, sem_wb))

        def fire_idx(j, par):
            iv, _, si, _, _ = bufs[par]
            pltpu.async_copy(idx.at[pl.ds(base + j * bb, bb)], iv, si)

        def proc(j, par, first, last):
            iv, rv, si, sg, sw = bufs[par]
            if not last:
                @pl.when(j + 1 < nb)
                def _():
                    fire_idx(j + 1, 1 - par)
            pltpu.make_async_copy(idx.at[pl.ds(base, bb)], iv, si).wait()
            if not first:
                @pl.when(j >= 2)
                def _():
                    pltpu.make_async_copy(
                        rv, out.at[pl.ds(base, bb)], sw).wait()
            descs = [
                pltpu.async_copy(table.at[iv.at[pl.ds(k * sub, sub)]],
                                 rv.at[pl.ds(k * sub, sub)], sg)
                for k in range(nsub)
            ]
            for d in descs:
                d.wait()
            pltpu.async_copy(rv, out.at[pl.ds(base + j * bb, bb)], sw)

        fire_idx(0, 0)

        def step(jj, carry):
            proc(2 * jj, 0, False, False)
            proc(2 * jj + 1, 1, False, False)
            return carry

        lax.fori_loop(0, nb2, step, 0)
        for par in (0, 1):
            _, rv, _, _, sw = bufs[par]
            pltpu.make_async_copy(rv, out.at[pl.ds(base, bb)], sw).wait()

    return pl.kernel(
        body,
        out_type=jax.ShapeDtypeStruct((Ep, D), jnp.float32),
        mesh=mesh,
        compiler_params=pltpu.CompilerParams(use_tc_tiling_on_sc=False),
        scratch_types=[
            pltpu.VMEM((bb,), jnp.int32),
            pltpu.VMEM((bb,), jnp.int32),
            pltpu.VMEM((bb, D), jnp.float32),
            pltpu.VMEM((bb, D), jnp.float32),
        ] + [pltpu.SemaphoreType.DMA] * 6,
    )


def _gather_rows(table, idx):
    D = table.shape[1]
    maxbb = 1024 if D <= 32 else 512
    return _gather_kernel(table.shape[0], idx.shape[0], D, maxbb)(table, idx)


@functools.cache
def _scatter_kernel(Ep, Tr, ranges):
    cnt = Ep // NS
    bb = _pick_block(cnt, maxbb=512)
    nsub = bb // 128
    nb = cnt // bb
    Tacc = Tr + 32
    wr = Tr // NS
    zr = Tacc // NS
    mesh = plsc.VectorSubcoreMesh(core_axis_name="c", subcore_axis_name="s")

    def body(vals, idx, zeros_hbm, out,
             idx_a, idx_b, midx_a, midx_b, vb_a, vb_b, acc,
             sem_ia, sem_ib, sem_va, sem_vb, sem_s):
        c = lax.axis_index("c")
        s = lax.axis_index("s")
        base = s * cnt
        bufs = ((idx_a, midx_a, vb_a, sem_ia, sem_va),
                (idx_b, midx_b, vb_b, sem_ib, sem_vb))

        def fire(j, par):
            iv, _, vv, si, sv = bufs[par]
            pltpu.async_copy(idx.at[pl.ds(base + j * bb, bb)], iv, si)
            pltpu.async_copy(
                vals.at[pl.ds(base + j * bb, bb), pl.ds(c * H, H)], vv, sv)

        for p in range(ranges):
            rbase = p * Tr
            pltpu.sync_copy(zeros_hbm.at[pl.ds(s * zr, zr)],
                            acc.at[pl.ds(s * zr, zr)])
            plsc.subcore_barrier()

            fire(0, 0)

            def proc(j, par):
                iv, mv, vv, si, sv = bufs[par]

                @pl.when(j + 1 < nb)
                def _():
                    fire(j + 1, 1 - par)

                pltpu.make_async_copy(
                    idx.at[pl.ds(base, bb)], iv, si).wait()
                for kk in range(bb // 16):
                    ivv = iv[pl.ds(kk * 16, 16)]
                    rel = ivv - rbase
                    ok = (rel >= 0) & (rel < Tr)
                    mv[kk // 8, pl.ds((kk % 8) * 16, 16)] = \
                        jnp.where(ok, rel, Tr)
                pltpu.make_async_copy(
                    vals.at[pl.ds(base, bb), pl.ds(c * H, H)], vv, sv).wait()
                descs = [
                    pltpu.async_copy(vv.at[pl.ds(k * 128, 128)],
                                     acc.at[mv.at[k]], sem_s, add=True)
                    for k in range(nsub)
                ]
                for d in descs:
                    d.wait()

            def step(jj, carry):
                proc(2 * jj, 0)
                proc(2 * jj + 1, 1)
                return carry

            lax.fori_loop(0, nb // 2, step, 0)
            plsc.subcore_barrier()
            pltpu.sync_copy(acc.at[pl.ds(s * wr, wr)],
                            out.at[pl.ds(rbase + s * wr, wr), pl.ds(c * H, H)])
            plsc.subcore_barrier()

    return pl.kernel(
        body,
        out_type=jax.ShapeDtypeStruct((Tr * ranges, F), jnp.float32),
        mesh=mesh,
        compiler_params=pltpu.CompilerParams(use_tc_tiling_on_sc=False),
        scratch_types=[
            pltpu.VMEM((bb,), jnp.int32),
            pltpu.VMEM((bb,), jnp.int32),
            pltpu.VMEM((nsub, 128), jnp.int32),
            pltpu.VMEM((nsub, 128), jnp.int32),
            pltpu.VMEM((bb, H), jnp.float32),
            pltpu.VMEM((bb, H), jnp.float32),
            pltpu.VMEM_SHARED((Tacc, H), jnp.float32),
        ] + [pltpu.SemaphoreType.DMA] * 5,
    )


def _scatter_rows(vals, idx, Tr, ranges):
    zeros_hbm = jnp.zeros((Tr + 32, H), jnp.float32)
    return _scatter_kernel(idx.shape[0], Tr, ranges)(vals, idx, zeros_hbm)


def _init_body(gf_ref, inc_ref, wi_ref, bt_ref, wg_ref, b_ref, o_ref):
    ip = jnp.dot(inc_ref[...], wi_ref[...], preferred_element_type=jnp.float32)
    oh = (bt_ref[...] == lax.broadcasted_iota(jnp.int32, (1, 64), 1)
          ).astype(jnp.float32)
    y = (jnp.dot(gf_ref[...], wg_ref[...], preferred_element_type=jnp.float32)
         + jnp.dot(oh, ip, preferred_element_type=jnp.float32) + b_ref[...])
    o_ref[...] = jnp.maximum(y, 0.0)


@functools.cache
def _tc_init_kernel(T, blk):
    grid = T // blk
    return pl.pallas_call(
        _init_body,
        grid=(grid,),
        in_specs=[
            pl.BlockSpec((blk, 8), lambda i: (i, 0)),
            pl.BlockSpec((64, 8), lambda i: (0, 0)),
            pl.BlockSpec((8, F), lambda i: (0, 0)),
            pl.BlockSpec((blk, 1), lambda i: (i, 0)),
            pl.BlockSpec((8, F), lambda i: (0, 0)),
            pl.BlockSpec((1, F), lambda i: (0, 0)),
        ],
        out_specs=pl.BlockSpec((blk, F), lambda i: (i, 0)),
        out_shape=jax.ShapeDtypeStruct((T, F), jnp.float32),
    )


def _qkvr_body(x_ref, wq, wkv, wr, b_ref, q_ref, kv_ref, r_ref, mq_ref, mk_ref):
    i = pl.program_id(0)
    x = x_ref[...]
    b = b_ref[...]
    q = jnp.dot(x, wq[...], preferred_element_type=jnp.float32) + b[:, 0:F]
    kv = jnp.dot(x, wkv[...], preferred_element_type=jnp.float32) + b[:, F:3 * F]
    q_ref[...] = q
    kv_ref[...] = kv
    r_ref[...] = jnp.dot(x, wr[...], preferred_element_type=jnp.float32) + b[:, 3 * F:4 * F]
    mq = jnp.max(jnp.sum(q * q, axis=1))
    mk = jnp.max(jnp.sum(kv[:, 0:F] * kv[:, 0:F], axis=1))

    @pl.when(i == 0)
    def _():
        mq_ref[0, 0] = mq
        mk_ref[0, 0] = mk

    @pl.when(i > 0)
    def _():
        mq_ref[0, 0] = jnp.maximum(mq_ref[0, 0], mq)
        mk_ref[0, 0] = jnp.maximum(mk_ref[0, 0], mk)


@functools.cache
def _tc_qkvr_kernel(T, Fin, blk):
    grid = T // blk
    sc = jax.ShapeDtypeStruct((1, 1), jnp.float32)
    return pl.pallas_call(
        _qkvr_body,
        grid=(grid,),
        in_specs=[pl.BlockSpec((blk, Fin), lambda i: (i, 0)),
                  pl.BlockSpec((Fin, F), lambda i: (0, 0)),
                  pl.BlockSpec((Fin, 2 * F), lambda i: (0, 0)),
                  pl.BlockSpec((Fin, F), lambda i: (0, 0)),
                  pl.BlockSpec((1, 4 * F), lambda i: (0, 0))],
        out_specs=[pl.BlockSpec((blk, F), lambda i: (i, 0)),
                   pl.BlockSpec((blk, 2 * F), lambda i: (i, 0)),
                   pl.BlockSpec((blk, F), lambda i: (i, 0)),
                   pl.BlockSpec(memory_space=pltpu.SMEM),
                   pl.BlockSpec(memory_space=pltpu.SMEM)],
        out_shape=[jax.ShapeDtypeStruct((T, F), jnp.float32),
                   jax.ShapeDtypeStruct((T, 2 * F), jnp.float32),
                   jax.ShapeDtypeStruct((T, F), jnp.float32), sc, sc],
    )


def _exws_body(qd_ref, kv_ref, m_ref, w_ref):
    kv = kv_ref[...]
    a = jnp.sum(qd_ref[...] * kv[:, 0:F], axis=1, keepdims=True) * SCALE
    ex = jnp.exp(a - m_ref[0, 0])
    col = lax.broadcasted_iota(jnp.int32, w_ref.shape, 1)
    w_ref[...] = kv[:, F:2 * F] * ex + jnp.where(col == F - 1, ex, 0.0)


@functools.cache
def _tc_exws_kernel(Ep, blk):
    grid = Ep // blk
    return pl.pallas_call(
        _exws_body,
        grid=(grid,),
        in_specs=[pl.BlockSpec((blk, F), lambda i: (i, 0)),
                  pl.BlockSpec((blk, 2 * F), lambda i: (i, 0)),
                  pl.BlockSpec(memory_space=pltpu.SMEM)],
        out_specs=pl.BlockSpec((blk, F), lambda i: (i, 0)),
        out_shape=jax.ShapeDtypeStruct((Ep, F), jnp.float32),
    )


def _combine_body(us_ref, r_ref, w1, w2, w3, o_ref):
    u = us_ref[...]
    s = u[:, F - 1:F]
    col = lax.broadcasted_iota(jnp.int32, u.shape, 1)
    out = jnp.where((col >= F - 2) | (s <= 0.0), 0.0, u / s)
    r = r_ref[...]
    lg = jnp.sum(out * w1[...] + r * w2[...] + (out - r) * w3[...],
                 axis=1, keepdims=True)
    beta = 1.0 / (1.0 + jnp.exp(-lg))
    o_ref[...] = jnp.maximum(beta * r + (1.0 - beta) * out, 0.0)


def _combine_final_body(us_ref, r_ref, w1, w2, w3, wf, bf, o_ref):
    u = us_ref[...]
    s = u[:, F - 1:F]
    col = lax.broadcasted_iota(jnp.int32, u.shape, 1)
    out = jnp.where((col >= F - 2) | (s <= 0.0), 0.0, u / s)
    r = r_ref[...]
    lg = jnp.sum(out * w1[...] + r * w2[...] + (out - r) * w3[...],
                 axis=1, keepdims=True)
    beta = 1.0 / (1.0 + jnp.exp(-lg))
    x = jnp.maximum(beta * r + (1.0 - beta) * out, 0.0)
    y = jnp.dot(x, wf[...], preferred_element_type=jnp.float32) + bf[...]
    o_ref[...] = jnp.maximum(y, 0.0)


@functools.cache
def _tc_combine_kernel(T, blk, with_final):
    grid = T // blk
    specs = [pl.BlockSpec((blk, F), lambda i: (i, 0))] * 2 \
        + [pl.BlockSpec((1, F), lambda i: (0, 0))] * 3
    body = _combine_body
    if with_final:
        specs += [pl.BlockSpec((F, F), lambda i: (0, 0)),
                  pl.BlockSpec((1, F), lambda i: (0, 0))]
        body = _combine_final_body
    return pl.pallas_call(
        body,
        grid=(grid,),
        in_specs=specs,
        out_specs=pl.BlockSpec((blk, F), lambda i: (i, 0)),
        out_shape=jax.ShapeDtypeStruct((T, F), jnp.float32),
    )


def _scalemul_body(b_ref, v_ref, o_ref):
    o_ref[...] = b_ref[...] * v_ref[...]


@functools.cache
def _tc_scalemul_kernel(T, blk):
    return pl.pallas_call(
        _scalemul_body,
        grid=(T // blk,),
        in_specs=[pl.BlockSpec((blk, F), lambda i: (i, 0)),
                  pl.BlockSpec((blk, 1), lambda i: (i, 0))],
        out_specs=pl.BlockSpec((blk, F), lambda i: (i, 0)),
        out_shape=jax.ShapeDtypeStruct((T, F), jnp.float32),
    )


def _pool_body(x_ref, bt_ref, s_ref, c_ref):
    i = pl.program_id(0)

    @pl.when(i == 0)
    def _():
        s_ref[...] = jnp.zeros_like(s_ref)
        c_ref[...] = jnp.zeros_like(c_ref)

    oh = (bt_ref[...] == lax.broadcasted_iota(jnp.int32, (1, 64), 1)
          ).astype(jnp.float32)
    s_ref[...] += lax.dot_general(oh, x_ref[...], (((0,), (0,)), ((), ())),
                                  preferred_element_type=jnp.float32)
    c_ref[...] += lax.dot_general(oh, jnp.ones_like(bt_ref[...], jnp.float32),
                                  (((0,), (0,)), ((), ())),
                                  preferred_element_type=jnp.float32)


@functools.cache
def _tc_pool_kernel(T, blk):
    return pl.pallas_call(
        _pool_body,
        grid=(T // blk,),
        in_specs=[pl.BlockSpec((blk, F), lambda i: (i, 0)),
                  pl.BlockSpec((blk, 1), lambda i: (i, 0))],
        out_specs=[pl.BlockSpec((64, F), lambda i: (0, 0)),
                   pl.BlockSpec((64, 1), lambda i: (0, 0))],
        out_shape=[jax.ShapeDtypeStruct((64, F), jnp.float32),
                   jax.ShapeDtypeStruct((64, 1), jnp.float32)],
    )


def _head_body(s_ref, c_ref, w_ref, b_ref, o_ref):
    m = s_ref[...] / jnp.maximum(c_ref[...], 1.0)
    v = jnp.sum(m * w_ref[...], axis=1, keepdims=True) + b_ref[0, 0]
    o_ref[...] = jnp.tanh(v)


@functools.cache
def _tc_head_kernel():
    return pl.pallas_call(
        _head_body,
        grid=(1,),
        in_specs=[pl.BlockSpec((64, F), lambda i: (0, 0)),
                  pl.BlockSpec((64, 1), lambda i: (0, 0)),
                  pl.BlockSpec((1, F), lambda i: (0, 0)),
                  pl.BlockSpec((1, 1), lambda i: (0, 0))],
        out_specs=pl.BlockSpec((64, 1), lambda i: (0, 0)),
        out_shape=jax.ShapeDtypeStruct((64, 1), jnp.float32),
    )


def _pad_w(w):
    din, dout = w.shape
    if din > 32:
        h = din // 2
        top = jnp.pad(w[:h], ((0, 32 - h), (0, F - dout)))
        bot = jnp.pad(w[h:], ((0, 32 - (din - h)), (0, F - dout)))
        return jnp.concatenate([top, bot], axis=0)
    return jnp.pad(w, ((0, 32 - din), (0, F - dout)))


def _pad_vec(b):
    return jnp.pad(b, (0, F - b.shape[0])).reshape(1, F)


def _conv(p, x, src, dst_g, dst_s, Tr, ranges, eblk, final=None):
    T, Fin = x.shape
    blk = 2000 if T == 100000 else 2048
    wq, wk, wv, wr = (_pad_w(p[n]["w"]) for n in ("q", "k", "v", "skip"))
    wkv = jnp.concatenate([wk, wv], axis=1)
    bc = jnp.concatenate(
        [_pad_vec(p[n]["b"]) for n in ("q", "k", "v", "skip")], axis=1)
    q, kv, r, mq, mk = _tc_qkvr_kernel(T, Fin, blk)(x, wq, wkv, wr, bc)
    m = jnp.sqrt(mq * mk) * SCALE - 30.0
    qd = _gather_rows(q, dst_g)
    kvs = _gather_rows(kv, src)
    Ep = src.shape[0]
    ws = _tc_exws_kernel(Ep, eblk)(qd, kvs, m)
    us = _scatter_rows(ws, dst_s, Tr, ranges)[:T]
    bw = p["beta_w"][:, 0]
    w1 = _pad_vec(bw[0:30])
    w2 = _pad_vec(bw[30:60])
    w3 = _pad_vec(bw[60:90])
    if final is None:
        return _tc_combine_kernel(T, blk, False)(us, r, w1, w2, w3)
    wf, bf = final
    return _tc_combine_kernel(T, blk, True)(us, r, w1, w2, w3, wf, bf)


@jax.jit
def _impl(graph_features, income, bonus_values_normed, batch, graph_edges,
          bonus_nodes, bonus_edges, bonus_batch, bonus_mapping, params):
    N = graph_features.shape[0]
    NB = bonus_nodes.shape[0]
    NBON = bonus_values_normed.shape[0]
    M = bonus_mapping.shape[1]
    E = graph_edges.shape[1]
    EB = bonus_edges.shape[1]
    Epad = 1638400
    EBpad = 819200
    NBpad = 212992
    BIG = 1 << 30
    p = params

    gf8 = jnp.pad(graph_features, ((0, 0), (0, 3)))
    inc8 = jnp.pad(income, ((0, 0), (0, 6)))
    wg = jnp.pad(p["init"]["w"][:5], ((0, 3), (0, 2)))
    wi = jnp.pad(p["init"]["w"][5:7], ((0, 6), (0, 2)))
    bi = _pad_vec(p["init"]["b"])
    bt2 = batch.astype(jnp.int32).reshape(N, 1)
    x = _tc_init_kernel(N, 2000)(gf8, inc8, wi, bt2, wg, bi)

    src = jnp.pad(graph_edges[0].astype(jnp.int32), (0, Epad - E))
    dst_g = jnp.pad(graph_edges[1].astype(jnp.int32), (0, Epad - E))
    dst_s = jnp.pad(graph_edges[1].astype(jnp.int32), (0, Epad - E),
                    constant_values=BIG)
    x = _conv(p["g1"], x, src, dst_g, dst_s, N, 1, 6400)
    x = _conv(p["g2"], x, src, dst_g, dst_s, N, 1, 6400)

    bn = jnp.pad(bonus_nodes.astype(jnp.int32), (0, NBpad - NB))
    xb = _gather_rows(x, bn)
    bsrc = jnp.pad(bonus_edges[0].astype(jnp.int32), (0, EBpad - EB))
    bdst_g = jnp.pad(bonus_edges[1].astype(jnp.int32), (0, EBpad - EB))
    bdst_s = jnp.pad(bonus_edges[1].astype(jnp.int32), (0, EBpad - EB),
                     constant_values=BIG)
    xb = _conv(p["b1"], xb, bsrc, bdst_g, bdst_s, NBpad // 2, 2, 6400)
    bb = jnp.pad(bonus_batch.astype(jnp.int32), (0, NBpad - NB),
                 constant_values=NBON)
    pooled = _scatter_rows(xb, bb, NBON, 1)
    c2 = _tc_scalemul_kernel(NBON, 2000)(
        pooled, bonus_values_normed.reshape(NBON, 1))
    cols = jnp.pad(bonus_mapping[1].astype(jnp.int32), (0, NBpad - M))
    rows = jnp.pad(bonus_mapping[0].astype(jnp.int32), (0, NBpad - M),
                   constant_values=BIG)
    gs = _gather_rows(c2, cols)
    bn20 = _scatter_rows(gs, rows, NBON, 1)
    bnode = jnp.concatenate(
        [bn20, jnp.zeros((N - NBON, F), jnp.float32)], axis=0)

    x3 = jnp.concatenate([x, bnode], axis=1)
    wf1 = _pad_w(p["final1"]["w"])
    bf1 = _pad_vec(p["final1"]["b"])
    x4 = _conv(p["g3"], x3, src, dst_g, dst_s, N, 1, 6400, final=(wf1, bf1))

    sums, counts = _tc_pool_kernel(N, 2000)(x4, bt2)
    w2v = _pad_vec(p["final2"]["w"][:, 0])
    b2s = p["final2"]["b"].reshape(1, 1)
    out = _tc_head_kernel()(sums, counts, w2v, b2s)
    return out.reshape(-1)


def kernel(graph_features, income, bonus_values_normed, batch, graph_edges,
           bonus_nodes, bonus_edges, bonus_batch, bonus_mapping, params):
    return _impl(graph_features, income, bonus_values_normed, batch,
                 graph_edges, bonus_nodes, bonus_edges, bonus_batch,
                 bonus_mapping, params)
